# Initial kernel scaffold; baseline (speedup 1.0000x reference)
import jax
import jax.numpy as jnp
from jax.experimental import pallas as pl


def kernel(x, edge_index, W1, b1, W2, b2, W3, b3, Wc, bc):
    raise NotImplementedError("write your pallas kernel here")


# speedup vs baseline: 44.6553x; 44.6553x over previous
import functools

import jax
import jax.numpy as jnp
from jax import lax
from jax.experimental import pallas as pl
from jax.experimental.pallas import tpu as pltpu
from jax.experimental.pallas import tpu_sc as plsc

N = 10000
E = 320000
NC = 2
NS = 16
NW = NC * NS
NP = 10240
RPW = NP // NS
WIN = 128
CHROWS = 16
EP = 327680
SPW = EP // (NW * WIN)
NCHUNK = SPW // CHROWS

_mesh = plsc.VectorSubcoreMesh(
    core_axis_name="c", subcore_axis_name="s", num_cores=NC, num_subcores=NS
)
_sc_params = pltpu.CompilerParams(use_tc_tiling_on_sc=False)


def _deg_body(dst_ref, zeros_ref, out_ref, idx_v, ones_v, deg_sh):
    c = lax.axis_index("c")
    s = lax.axis_index("s")
    w = c * NS + s

    @pl.when(s == 0)
    def _():
        pltpu.sync_copy(zeros_ref, deg_sh)

    for k in range(WIN // 16):
        ones_v[pl.ds(k * 16, 16)] = jnp.ones((16,), jnp.float32)
    plsc.subcore_barrier()

    def chunk(i, carry):
        r0 = w * SPW + i * CHROWS
        pltpu.sync_copy(dst_ref.at[pl.ds(r0, CHROWS)], idx_v)
        for j in range(CHROWS):
            pltpu.sync_copy(ones_v, deg_sh.at[idx_v.at[j]], add=True)
        return carry

    lax.fori_loop(0, NCHUNK, chunk, 0)
    plsc.subcore_barrier()
    pltpu.sync_copy(deg_sh.at[pl.ds(s * RPW, RPW)], out_ref.at[c, pl.ds(s * RPW, RPW)])


def _mp_body(y_ref, src_ref, dst_ref, zeros_ref, out_ref,
             isrc_v, idst_v, rows_v, y_sh, acc_sh):
    c = lax.axis_index("c")
    s = lax.axis_index("s")
    w = c * NS + s

    @pl.when(s == 0)
    def _():
        pltpu.sync_copy(y_ref, y_sh)

    @pl.when(s == 1)
    def _():
        pltpu.sync_copy(zeros_ref, acc_sh)

    plsc.subcore_barrier()

    def chunk(i, carry):
        r0 = w * SPW + i * CHROWS
        pltpu.sync_copy(src_ref.at[pl.ds(r0, CHROWS)], isrc_v)
        pltpu.sync_copy(dst_ref.at[pl.ds(r0, CHROWS)], idst_v)
        for j in range(CHROWS):
            pltpu.sync_copy(y_sh.at[isrc_v.at[j]], rows_v.at[pl.ds(j * WIN, WIN)])
            pltpu.sync_copy(rows_v.at[pl.ds(j * WIN, WIN)],
                            acc_sh.at[idst_v.at[j]], add=True)
        return carry

    lax.fori_loop(0, NCHUNK, chunk, 0)
    plsc.subcore_barrier()
    pltpu.sync_copy(acc_sh.at[pl.ds(s * RPW, RPW)],
                    out_ref.at[c, pl.ds(s * RPW, RPW)])


def _deg_call(dst2d, zeros1):
    return pl.kernel(
        _deg_body,
        out_type=jax.ShapeDtypeStruct((NC, NP), jnp.float32),
        mesh=_mesh,
        compiler_params=_sc_params,
        scratch_types=[
            pltpu.VMEM((CHROWS, WIN), jnp.int32),
            pltpu.VMEM((WIN,), jnp.float32),
            pltpu.VMEM_SHARED((NP,), jnp.float32),
        ],
    )(dst2d, zeros1)


def _mp_call(y, src2d, dst2d, zerosf, f):
    return pl.kernel(
        _mp_body,
        out_type=jax.ShapeDtypeStruct((NC, NP, f), jnp.float32),
        mesh=_mesh,
        compiler_params=_sc_params,
        scratch_types=[
            pltpu.VMEM((CHROWS, WIN), jnp.int32),
            pltpu.VMEM((CHROWS, WIN), jnp.int32),
            pltpu.VMEM((CHROWS * WIN, f), jnp.float32),
            pltpu.VMEM_SHARED((NP, f), jnp.float32),
            pltpu.VMEM_SHARED((NP, f), jnp.float32),
        ],
    )(y, src2d, dst2d, zerosf)


def _small_mm(h, w_ref, fin):
    acc = h[:, 0:1] * w_ref[0:1, :]
    for k in range(1, fin):
        acc = acc + h[:, k : k + 1] * w_ref[k : k + 1, :]
    return acc


def _tca_body(x_ref, w1_ref, degp_ref, y_ref, dis_ref):
    deg = degp_ref[0] + degp_ref[1] + 1.0
    dis = lax.rsqrt(deg)
    dis = dis * (1.5 - 0.5 * deg * dis * dis)
    xw = jnp.dot(x_ref[...], w1_ref[...], preferred_element_type=jnp.float32)
    y_ref[...] = xw * dis
    dis_ref[...] = dis


def _tca(x_pad, w1p, degp2):
    return pl.pallas_call(
        _tca_body,
        out_shape=[
            jax.ShapeDtypeStruct((NP, 8), jnp.float32),
            jax.ShapeDtypeStruct((NP, 1), jnp.float32),
        ],
    )(x_pad, w1p, degp2)


def _tcb_body(fin, accp_ref, y_ref, dis_ref, b_ref, w_ref, ynext_ref):
    dis = dis_ref[...]
    acc = accp_ref[0] + accp_ref[1] + y_ref[...]
    h = jnp.tanh(acc * dis + b_ref[...])
    ynext_ref[...] = _small_mm(h, w_ref[...], fin) * dis


def _tcb(accp, y, dis, b2d, wp, fin):
    return pl.pallas_call(
        functools.partial(_tcb_body, fin),
        out_shape=jax.ShapeDtypeStruct((NP, 8), jnp.float32),
    )(accp, y, dis, b2d, wp)


def _tcd_body(accp_ref, y_ref, dis_ref, b_ref, wc_ref, bc_ref, out_ref, h_ref):
    dis = dis_ref[...]
    acc = accp_ref[0] + accp_ref[1] + y_ref[...]
    h = jnp.tanh(acc * dis + b_ref[...])
    h_ref[...] = h[:, 0:2]
    out_ref[...] = _small_mm(h, wc_ref[...], 2) + bc_ref[...]


def _tcd(accp, y, dis, b2d, wc, bc2d):
    return pl.pallas_call(
        _tcd_body,
        out_shape=[
            jax.ShapeDtypeStruct((NP, 4), jnp.float32),
            jax.ShapeDtypeStruct((NP, 2), jnp.float32),
        ],
    )(accp, y, dis, b2d, wc, bc2d)


def kernel(x, edge_index, W1, b1, W2, b2, W3, b3, Wc, bc):
    src = edge_index[0].astype(jnp.int32)
    dst = edge_index[1].astype(jnp.int32)
    npad = EP - E
    pad_idx = (jnp.arange(npad, dtype=jnp.int32) % (NP - N)) + N
    src2d = jnp.concatenate([src, pad_idx]).reshape(EP // WIN, WIN)
    dst2d = jnp.concatenate([dst, pad_idx]).reshape(EP // WIN, WIN)

    x_pad = jnp.pad(x, ((0, NP - N), (0, 0)))
    zeros1 = jnp.zeros((NP,), jnp.float32)
    zeros8 = jnp.zeros((NP, 8), jnp.float32)
    w1p = jnp.pad(W1, ((0, 0), (0, 4)))
    w2p = jnp.pad(W2, ((0, 0), (0, 4)))
    w3p = jnp.pad(W3, ((0, 0), (0, 6)))
    b1p = jnp.pad(b1, (0, 4)).reshape(1, 8)
    b2p = jnp.pad(b2, (0, 4)).reshape(1, 8)
    b3p = jnp.pad(b3, (0, 6)).reshape(1, 8)

    degp = _deg_call(dst2d, zeros1)
    y1, dis = _tca(x_pad, w1p, degp[..., None])

    acc1 = _mp_call(y1, src2d, dst2d, zeros8, 8)
    y2 = _tcb(acc1, y1, dis, b1p, w2p, 4)

    acc2 = _mp_call(y2, src2d, dst2d, zeros8, 8)
    y3 = _tcb(acc2, y2, dis, b2p, w3p, 4)

    acc3 = _mp_call(y3, src2d, dst2d, zeros8, 8)
    out_full, h_full = _tcd(acc3, y3, dis, b3p, Wc, bc.reshape(1, 4))

    return (out_full[:N], h_full[:N])


# speedup vs baseline: 52.6704x; 1.1795x over previous
import functools

import jax
import jax.numpy as jnp
from jax import lax
from jax.experimental import pallas as pl
from jax.experimental.pallas import tpu as pltpu
from jax.experimental.pallas import tpu_sc as plsc

N = 10000
E = 320000
NC = 2
NS = 16
NP = 10240
RPT = NP // NS
GPT = RPT // 2
WIN = 128
NWIN = E // WIN
WPW = 78
CHROWS = 13
NCHUNK = WPW // CHROWS
NEXTRA = NWIN - 32 * WPW

_mesh = plsc.VectorSubcoreMesh(
    core_axis_name="c", subcore_axis_name="s", num_cores=NC, num_subcores=NS
)
_sc_params = pltpu.CompilerParams(use_tc_tiling_on_sc=False,
                                  needs_layout_passes=False)


def _tanh16(v):
    av = jnp.abs(v)
    e = jnp.exp(av * -2.0)
    t = (1.0 - e) / (1.0 + e)
    return jnp.where(v < 0.0, -t, t)


def _fill_dis(g_d0, g_d1, g_dis):
    def body(j, carry):
        d = g_d0[pl.ds(16 * j, 16)] + g_d1[pl.ds(16 * j, 16)] + 1.0
        i = plsc.bitcast(d, jnp.int32)
        i = 0x5F3759DF - lax.shift_right_logical(i, 1)
        r = plsc.bitcast(i, jnp.float32)
        r = r * (1.5 - 0.5 * d * r * r)
        r = r * (1.5 - 0.5 * d * r * r)
        r = r * (1.5 - 0.5 * d * r * r)
        g_dis[pl.ds(16 * j, 16)] = r
        return carry

    lax.fori_loop(0, RPT // 16, body, 0)


def _edge_pass(w, ei_ref, y_sh, acc_sh, isrc_v, idst_v, rows_v):
    def chunk(i, carry):
        r0 = w * WPW + i * CHROWS
        pltpu.sync_copy(ei_ref.at[0, pl.ds(r0, CHROWS)], isrc_v)
        pltpu.sync_copy(ei_ref.at[1, pl.ds(r0, CHROWS)], idst_v)
        for j in range(CHROWS):
            pltpu.sync_copy(y_sh.at[isrc_v.at[j]], rows_v.at[pl.ds(j * WIN, WIN)])
            pltpu.sync_copy(rows_v.at[pl.ds(j * WIN, WIN)],
                            acc_sh.at[idst_v.at[j]], add=True)
        return carry

    lax.fori_loop(0, NCHUNK, chunk, 0)

    @pl.when(w < NEXTRA)
    def _():
        r0 = 32 * WPW + w
        pltpu.sync_copy(ei_ref.at[0, pl.ds(r0, 1)], isrc_v.at[pl.ds(0, 1)])
        pltpu.sync_copy(ei_ref.at[1, pl.ds(r0, 1)], idst_v.at[pl.ds(0, 1)])
        pltpu.sync_copy(y_sh.at[isrc_v.at[0]], rows_v.at[pl.ds(0, WIN)])
        pltpu.sync_copy(rows_v.at[pl.ds(0, WIN)], acc_sh.at[idst_v.at[0]],
                        add=True)


def _deg_body(ei_ref, zeros_ref, out_ref, idx_v, ones_v, deg_sh):
    c = lax.axis_index("c")
    s = lax.axis_index("s")
    w = c * NS + s

    @pl.when(s == 0)
    def _():
        pltpu.sync_copy(zeros_ref, deg_sh)

    for k in range(WIN // 16):
        ones_v[pl.ds(k * 16, 16)] = jnp.ones((16,), jnp.float32)
    plsc.subcore_barrier()

    def chunk(i, carry):
        r0 = w * WPW + i * CHROWS
        pltpu.sync_copy(ei_ref.at[1, pl.ds(r0, CHROWS)], idx_v)
        for j in range(CHROWS):
            pltpu.sync_copy(ones_v, deg_sh.at[idx_v.at[j]], add=True)
        return carry

    lax.fori_loop(0, NCHUNK, chunk, 0)

    @pl.when(w < NEXTRA)
    def _():
        r0 = 32 * WPW + w
        pltpu.sync_copy(ei_ref.at[1, pl.ds(r0, 1)], idx_v.at[pl.ds(0, 1)])
        pltpu.sync_copy(ones_v, deg_sh.at[idx_v.at[0]], add=True)

    plsc.subcore_barrier()
    pltpu.sync_copy(deg_sh.at[pl.ds(s * RPT, RPT)],
                    out_ref.at[c, pl.ds(s * RPT, RPT)])


def _deg_call(ei3, zeros1):
    return pl.kernel(
        _deg_body,
        out_type=jax.ShapeDtypeStruct((NC, NP), jnp.float32),
        mesh=_mesh,
        compiler_params=_sc_params,
        scratch_types=[
            pltpu.VMEM((CHROWS, WIN), jnp.int32),
            pltpu.VMEM((WIN,), jnp.float32),
            pltpu.VMEM_SHARED((NP,), jnp.float32),
        ],
    )(ei3, zeros1)


def _mp1_body(ei_ref, xw_ref, degp_ref, zeros_ref, out_ref,
              isrc_v, idst_v, rows_v, g_d0, g_d1, g_dis, g_in, g_y,
              y_sh, acc_sh):
    c = lax.axis_index("c")
    s = lax.axis_index("s")
    w = c * NS + s

    @pl.when(s == 0)
    def _():
        pltpu.sync_copy(zeros_ref, acc_sh)

    pltpu.sync_copy(degp_ref.at[0, pl.ds(s * RPT, RPT)], g_d0)
    pltpu.sync_copy(degp_ref.at[1, pl.ds(s * RPT, RPT)], g_d1)
    pltpu.sync_copy(xw_ref.at[pl.ds(s * RPT, RPT)], g_in)
    _fill_dis(g_d0, g_d1, g_dis)

    iota = lax.iota(jnp.int32, 16)
    rhalf = lax.shift_right_logical(iota, 3)
    col16 = jnp.bitwise_and(iota, 7)

    def grp(g, carry):
        rowi = rhalf + 2 * g
        db = plsc.load_gather(g_dis, (rowi,))
        xw = plsc.load_gather(g_in, (rowi, col16))
        plsc.store_scatter(g_y, (rowi, col16), db * xw)
        return carry

    lax.fori_loop(0, GPT, grp, 0)
    pltpu.sync_copy(g_y, y_sh.at[pl.ds(s * RPT, RPT)])
    plsc.subcore_barrier()
    _edge_pass(w, ei_ref, y_sh, acc_sh, isrc_v, idst_v, rows_v)
    plsc.subcore_barrier()
    pltpu.sync_copy(acc_sh.at[pl.ds(s * RPT, RPT)],
                    out_ref.at[c, pl.ds(s * RPT, RPT)])


def _mp1_call(ei3, xw1, degp, zeros8):
    return pl.kernel(
        _mp1_body,
        out_type=jax.ShapeDtypeStruct((NC, NP, 8), jnp.float32),
        mesh=_mesh,
        compiler_params=_sc_params,
        scratch_types=[
            pltpu.VMEM((CHROWS, WIN), jnp.int32),
            pltpu.VMEM((CHROWS, WIN), jnp.int32),
            pltpu.VMEM((CHROWS * WIN, 8), jnp.float32),
            pltpu.VMEM((RPT,), jnp.float32),
            pltpu.VMEM((RPT,), jnp.float32),
            pltpu.VMEM((RPT,), jnp.float32),
            pltpu.VMEM((RPT, 8), jnp.float32),
            pltpu.VMEM((RPT, 8), jnp.float32),
            pltpu.VMEM_SHARED((NP, 8), jnp.float32),
            pltpu.VMEM_SHARED((NP, 8), jnp.float32),
        ],
    )(ei3, xw1, degp, zeros8)


def _glue_layer(g_a0, g_a1, g_prev, g_dis, g_c, g_y, g_t, g_dis8=None,
                scale_prev=False):
    iota = lax.iota(jnp.int32, 16)
    rhalf = lax.shift_right_logical(iota, 3)
    col16 = jnp.bitwise_and(iota, 7)
    rh8 = rhalf * 8
    bv = g_c[0]

    def grp(g, carry):
        rowi = rhalf + 2 * g
        db = plsc.load_gather(g_dis, (rowi,))
        prev = plsc.load_gather(g_prev, (rowi, col16))
        if scale_prev:
            prev = prev * db
        a = (plsc.load_gather(g_a0, (rowi, col16))
             + plsc.load_gather(g_a1, (rowi, col16))
             + prev)
        t = _tanh16(a * db + bv)
        g_t[...] = t
        acc = plsc.load_gather(g_t, (rh8,)) * g_c[1]
        acc = acc + plsc.load_gather(g_t, (rh8 + 1,)) * g_c[2]
        acc = acc + plsc.load_gather(g_t, (rh8 + 2,)) * g_c[3]
        acc = acc + plsc.load_gather(g_t, (rh8 + 3,)) * g_c[4]
        plsc.store_scatter(g_y, (rowi, col16), db * acc)
        if g_dis8 is not None:
            plsc.store_scatter(g_dis8, (rowi, col16), db)
        return carry

    lax.fori_loop(0, GPT, grp, 0)


def _mp2_body(ei_ref, accp_ref, prev_ref, degp_ref, cst_ref, zeros_ref,
              out_ref, ynext_ref,
              isrc_v, idst_v, rows_v, g_d0, g_d1, g_dis,
              g_a0, g_a1, g_prev, g_c, g_t, g_y, y_sh, acc_sh):
    c = lax.axis_index("c")
    s = lax.axis_index("s")
    w = c * NS + s

    @pl.when(s == 0)
    def _():
        pltpu.sync_copy(zeros_ref, acc_sh)

    pltpu.sync_copy(degp_ref.at[0, pl.ds(s * RPT, RPT)], g_d0)
    pltpu.sync_copy(degp_ref.at[1, pl.ds(s * RPT, RPT)], g_d1)
    pltpu.sync_copy(accp_ref.at[0, pl.ds(s * RPT, RPT)], g_a0)
    pltpu.sync_copy(accp_ref.at[1, pl.ds(s * RPT, RPT)], g_a1)
    pltpu.sync_copy(prev_ref.at[pl.ds(s * RPT, RPT)], g_prev)
    pltpu.sync_copy(cst_ref, g_c)
    _fill_dis(g_d0, g_d1, g_dis)
    _glue_layer(g_a0, g_a1, g_prev, g_dis, g_c, g_y, g_t, scale_prev=True)
    pltpu.sync_copy(g_y, y_sh.at[pl.ds(s * RPT, RPT)])

    @pl.when(c == 0)
    def _():
        pltpu.sync_copy(g_y, ynext_ref.at[pl.ds(s * RPT, RPT)])

    plsc.subcore_barrier()
    _edge_pass(w, ei_ref, y_sh, acc_sh, isrc_v, idst_v, rows_v)
    plsc.subcore_barrier()
    pltpu.sync_copy(acc_sh.at[pl.ds(s * RPT, RPT)],
                    out_ref.at[c, pl.ds(s * RPT, RPT)])


def _mp2_call(ei3, accp, prev_y, degp, cst, zeros8):
    return pl.kernel(
        _mp2_body,
        out_type=[
            jax.ShapeDtypeStruct((NC, NP, 8), jnp.float32),
            jax.ShapeDtypeStruct((NP, 8), jnp.float32),
        ],
        mesh=_mesh,
        compiler_params=_sc_params,
        scratch_types=[
            pltpu.VMEM((CHROWS, WIN), jnp.int32),
            pltpu.VMEM((CHROWS, WIN), jnp.int32),
            pltpu.VMEM((CHROWS * WIN, 8), jnp.float32),
            pltpu.VMEM((RPT,), jnp.float32),
            pltpu.VMEM((RPT,), jnp.float32),
            pltpu.VMEM((RPT,), jnp.float32),
            pltpu.VMEM((RPT, 8), jnp.float32),
            pltpu.VMEM((RPT, 8), jnp.float32),
            pltpu.VMEM((RPT, 8), jnp.float32),
            pltpu.VMEM((5, 16), jnp.float32),
            pltpu.VMEM((16,), jnp.float32),
            pltpu.VMEM((RPT, 8), jnp.float32),
            pltpu.VMEM_SHARED((NP, 8), jnp.float32),
            pltpu.VMEM_SHARED((NP, 8), jnp.float32),
        ],
    )(ei3, accp, prev_y, degp, cst, zeros8)


def _mp3_body(ei_ref, accp_ref, prev_ref, degp_ref, cst_ref, zeros_ref,
              out_ref, ynext_ref, dis8_ref,
              isrc_v, idst_v, rows_v, g_d0, g_d1, g_dis,
              g_a0, g_a1, g_prev, g_c, g_t, g_y, g_dis8, y_sh, acc_sh):
    c = lax.axis_index("c")
    s = lax.axis_index("s")
    w = c * NS + s

    @pl.when(s == 0)
    def _():
        pltpu.sync_copy(zeros_ref, acc_sh)

    pltpu.sync_copy(degp_ref.at[0, pl.ds(s * RPT, RPT)], g_d0)
    pltpu.sync_copy(degp_ref.at[1, pl.ds(s * RPT, RPT)], g_d1)
    pltpu.sync_copy(accp_ref.at[0, pl.ds(s * RPT, RPT)], g_a0)
    pltpu.sync_copy(accp_ref.at[1, pl.ds(s * RPT, RPT)], g_a1)
    pltpu.sync_copy(prev_ref.at[pl.ds(s * RPT, RPT)], g_prev)
    pltpu.sync_copy(cst_ref, g_c)
    _fill_dis(g_d0, g_d1, g_dis)
    _glue_layer(g_a0, g_a1, g_prev, g_dis, g_c, g_y, g_t, g_dis8=g_dis8)
    pltpu.sync_copy(g_y, y_sh.at[pl.ds(s * RPT, RPT)])

    @pl.when(c == 0)
    def _():
        pltpu.sync_copy(g_y, ynext_ref.at[pl.ds(s * RPT, RPT)])
        pltpu.sync_copy(g_dis8, dis8_ref.at[pl.ds(s * RPT, RPT)])

    plsc.subcore_barrier()
    _edge_pass(w, ei_ref, y_sh, acc_sh, isrc_v, idst_v, rows_v)
    plsc.subcore_barrier()
    pltpu.sync_copy(acc_sh.at[pl.ds(s * RPT, RPT)],
                    out_ref.at[c, pl.ds(s * RPT, RPT)])


def _mp3_call(ei3, accp, prev_y, degp, cst, zeros8):
    return pl.kernel(
        _mp3_body,
        out_type=[
            jax.ShapeDtypeStruct((NC, NP, 8), jnp.float32),
            jax.ShapeDtypeStruct((NP, 8), jnp.float32),
            jax.ShapeDtypeStruct((NP, 8), jnp.float32),
        ],
        mesh=_mesh,
        compiler_params=_sc_params,
        scratch_types=[
            pltpu.VMEM((CHROWS, WIN), jnp.int32),
            pltpu.VMEM((CHROWS, WIN), jnp.int32),
            pltpu.VMEM((CHROWS * WIN, 8), jnp.float32),
            pltpu.VMEM((RPT,), jnp.float32),
            pltpu.VMEM((RPT,), jnp.float32),
            pltpu.VMEM((RPT,), jnp.float32),
            pltpu.VMEM((RPT, 8), jnp.float32),
            pltpu.VMEM((RPT, 8), jnp.float32),
            pltpu.VMEM((RPT, 8), jnp.float32),
            pltpu.VMEM((5, 16), jnp.float32),
            pltpu.VMEM((16,), jnp.float32),
            pltpu.VMEM((RPT, 8), jnp.float32),
            pltpu.VMEM((RPT, 8), jnp.float32),
            pltpu.VMEM_SHARED((NP, 8), jnp.float32),
            pltpu.VMEM_SHARED((NP, 8), jnp.float32),
        ],
    )(ei3, accp, prev_y, degp, cst, zeros8)


def _tca_body(x_ref, w1_ref, y_ref):
    y_ref[...] = jnp.dot(x_ref[...], w1_ref[...],
                         preferred_element_type=jnp.float32)


def _tca(x_pad, w1p):
    return pl.pallas_call(
        _tca_body,
        out_shape=jax.ShapeDtypeStruct((NP, 8), jnp.float32),
    )(x_pad, w1p)


def _tcd_body(accp_ref, y_ref, dis8_ref, b_ref, wc_ref, bc_ref, out_ref, h_ref):
    dis8 = dis8_ref[...]
    a = (accp_ref[0] + accp_ref[1] + y_ref[...]) * dis8 + b_ref[...]
    h = jnp.tanh(a)
    h_ref[...] = h[:, 0:2]
    out_ref[...] = (h[:, 0:1] * wc_ref[0:1, :] + h[:, 1:2] * wc_ref[1:2, :]
                    + bc_ref[...])


def _tcd(accp, y, dis8, b2d, wc, bc2d):
    return pl.pallas_call(
        _tcd_body,
        out_shape=[
            jax.ShapeDtypeStruct((NP, 4), jnp.float32),
            jax.ShapeDtypeStruct((NP, 2), jnp.float32),
        ],
    )(accp, y, dis8, b2d, wc, bc2d)


def _const_block(b, w):
    bp = jnp.pad(b, (0, 8 - b.shape[0]))
    wp = jnp.pad(w, ((0, 4 - w.shape[0]), (0, 8 - w.shape[1])))
    rows = [jnp.tile(bp, 2)] + [jnp.tile(wp[k], 2) for k in range(4)]
    return jnp.stack(rows)


def kernel(x, edge_index, W1, b1, W2, b2, W3, b3, Wc, bc):
    ei3 = edge_index.astype(jnp.int32).reshape(2, NWIN, WIN)
    x_pad = jnp.pad(x, ((0, NP - N), (0, 0)))
    w1p = jnp.pad(W1, ((0, 0), (0, 4)))
    zeros1 = jnp.zeros((NP,), jnp.float32)
    zeros8 = jnp.zeros((NP, 8), jnp.float32)
    cst1 = _const_block(b1, W2)
    cst2 = _const_block(b2, W3)

    xw1 = _tca(x_pad, w1p)
    degp = _deg_call(ei3, zeros1)

    acc1 = _mp1_call(ei3, xw1, degp, zeros8)
    acc2, y2 = _mp2_call(ei3, acc1, xw1, degp, cst1, zeros8)
    acc3, y3, dis8 = _mp3_call(ei3, acc2, y2, degp, cst2, zeros8)

    out_full, h_full = _tcd(acc3, y3, dis8, jnp.pad(b3, (0, 6)).reshape(1, 8),
                            Wc, bc.reshape(1, 4))
    return (out_full[:N], h_full[:N])


# speedup vs baseline: 61.1128x; 1.1603x over previous
import functools

import jax
import jax.numpy as jnp
from jax import lax
from jax.experimental import pallas as pl
from jax.experimental.pallas import tpu as pltpu
from jax.experimental.pallas import tpu_sc as plsc

N = 10000
E = 320000
NC = 2
NS = 16
NP = 10240
RPT = NP // NS
GPT = RPT // 2
WIN = 128
NWIN = E // WIN
WPW = 78
CHROWS = 13
NCHUNK = WPW // CHROWS
NEXTRA = NWIN - 32 * WPW

_mesh = plsc.VectorSubcoreMesh(
    core_axis_name="c", subcore_axis_name="s", num_cores=NC, num_subcores=NS
)
_sc_params = pltpu.CompilerParams(use_tc_tiling_on_sc=False,
                                  needs_layout_passes=False)


def _tanh16(v):
    av = jnp.abs(v)
    e = jnp.exp(av * -2.0)
    t = (1.0 - e) / (1.0 + e)
    return jnp.where(v < 0.0, -t, t)


def _fill_dis(g_d0, g_d1, g_dis):
    def body(j, carry):
        d = g_d0[pl.ds(16 * j, 16)] + g_d1[pl.ds(16 * j, 16)] + 1.0
        i = plsc.bitcast(d, jnp.int32)
        i = 0x5F3759DF - lax.shift_right_logical(i, 1)
        r = plsc.bitcast(i, jnp.float32)
        r = r * (1.5 - 0.5 * d * r * r)
        r = r * (1.5 - 0.5 * d * r * r)
        r = r * (1.5 - 0.5 * d * r * r)
        g_dis[pl.ds(16 * j, 16)] = r
        return carry

    lax.fori_loop(0, RPT // 16, body, 0, unroll=2)


def _edge_pass(w, ei_ref, y_sh, acc_sh, isrc_v, idst_v, rows_v,
               gsem_a, gsem_b, ssem):
    gsems = (gsem_a, gsem_b)

    def chunk(i, carry):
        r0 = w * WPW + i * CHROWS
        pltpu.sync_copy(ei_ref.at[0, pl.ds(r0, CHROWS)], isrc_v)
        pltpu.sync_copy(ei_ref.at[1, pl.ds(r0, CHROWS)], idst_v)
        gath = [None] * CHROWS
        gath[0] = pltpu.async_copy(y_sh.at[isrc_v.at[0]],
                                   rows_v.at[pl.ds(0, WIN)], gsems[0])
        scat = []
        for j in range(CHROWS):
            if j + 1 < CHROWS:
                gath[j + 1] = pltpu.async_copy(
                    y_sh.at[isrc_v.at[j + 1]],
                    rows_v.at[pl.ds((j + 1) * WIN, WIN)], gsems[(j + 1) % 2])
            gath[j].wait()
            scat.append(pltpu.async_copy(rows_v.at[pl.ds(j * WIN, WIN)],
                                         acc_sh.at[idst_v.at[j]], ssem,
                                         add=True))
        for d in scat:
            d.wait()
        return carry

    lax.fori_loop(0, NCHUNK, chunk, 0)

    @pl.when(w < NEXTRA)
    def _():
        r0 = 32 * WPW + w
        pltpu.sync_copy(ei_ref.at[0, pl.ds(r0, 1)], isrc_v.at[pl.ds(0, 1)])
        pltpu.sync_copy(ei_ref.at[1, pl.ds(r0, 1)], idst_v.at[pl.ds(0, 1)])
        pltpu.sync_copy(y_sh.at[isrc_v.at[0]], rows_v.at[pl.ds(0, WIN)])
        pltpu.sync_copy(rows_v.at[pl.ds(0, WIN)], acc_sh.at[idst_v.at[0]],
                        add=True)


def _deg_body(ei_ref, zeros_ref, out_ref, idx_v, ones_v, deg_sh):
    c = lax.axis_index("c")
    s = lax.axis_index("s")
    w = c * NS + s

    @pl.when(s == 0)
    def _():
        pltpu.sync_copy(zeros_ref, deg_sh)

    for k in range(WIN // 16):
        ones_v[pl.ds(k * 16, 16)] = jnp.ones((16,), jnp.float32)
    plsc.subcore_barrier()

    def chunk(i, carry):
        r0 = w * WPW + i * CHROWS
        pltpu.sync_copy(ei_ref.at[1, pl.ds(r0, CHROWS)], idx_v)
        for j in range(CHROWS):
            pltpu.sync_copy(ones_v, deg_sh.at[idx_v.at[j]], add=True)
        return carry

    lax.fori_loop(0, NCHUNK, chunk, 0)

    @pl.when(w < NEXTRA)
    def _():
        r0 = 32 * WPW + w
        pltpu.sync_copy(ei_ref.at[1, pl.ds(r0, 1)], idx_v.at[pl.ds(0, 1)])
        pltpu.sync_copy(ones_v, deg_sh.at[idx_v.at[0]], add=True)

    plsc.subcore_barrier()
    pltpu.sync_copy(deg_sh.at[pl.ds(s * RPT, RPT)],
                    out_ref.at[c, pl.ds(s * RPT, RPT)])


def _deg_call(ei3, zeros1):
    return pl.kernel(
        _deg_body,
        out_type=jax.ShapeDtypeStruct((NC, NP), jnp.float32),
        mesh=_mesh,
        compiler_params=_sc_params,
        scratch_types=[
            pltpu.VMEM((CHROWS, WIN), jnp.int32),
            pltpu.VMEM((WIN,), jnp.float32),
            pltpu.VMEM_SHARED((NP,), jnp.float32),
        ],
    )(ei3, zeros1)


def _mp1_body(ei_ref, xw_ref, degp_ref, zeros_ref, out_ref,
              isrc_v, idst_v, rows_v, g_d0, g_d1, g_dis, g_in, g_y,
              y_sh, acc_sh, gsem_a, gsem_b, ssem):
    c = lax.axis_index("c")
    s = lax.axis_index("s")
    w = c * NS + s

    @pl.when(s == 0)
    def _():
        pltpu.sync_copy(zeros_ref, acc_sh)

    pltpu.sync_copy(degp_ref.at[0, pl.ds(s * RPT, RPT)], g_d0)
    pltpu.sync_copy(degp_ref.at[1, pl.ds(s * RPT, RPT)], g_d1)
    pltpu.sync_copy(xw_ref.at[pl.ds(s * RPT, RPT)], g_in)
    _fill_dis(g_d0, g_d1, g_dis)

    iota = lax.iota(jnp.int32, 16)
    rhalf = lax.shift_right_logical(iota, 3)
    col16 = jnp.bitwise_and(iota, 7)

    def grp(g, carry):
        rowi = rhalf + 2 * g
        db = plsc.load_gather(g_dis, (rowi,))
        xw = plsc.load_gather(g_in, (rowi, col16))
        plsc.store_scatter(g_y, (rowi, col16), db * xw)
        return carry

    lax.fori_loop(0, GPT, grp, 0, unroll=4)
    pltpu.sync_copy(g_y, y_sh.at[pl.ds(s * RPT, RPT)])
    plsc.subcore_barrier()
    _edge_pass(w, ei_ref, y_sh, acc_sh, isrc_v, idst_v, rows_v, gsem_a, gsem_b, ssem)
    plsc.subcore_barrier()
    pltpu.sync_copy(acc_sh.at[pl.ds(s * RPT, RPT)],
                    out_ref.at[c, pl.ds(s * RPT, RPT)])


def _mp1_call(ei3, xw1, degp, zeros8):
    return pl.kernel(
        _mp1_body,
        out_type=jax.ShapeDtypeStruct((NC, NP, 8), jnp.float32),
        mesh=_mesh,
        compiler_params=_sc_params,
        scratch_types=[
            pltpu.VMEM((CHROWS, WIN), jnp.int32),
            pltpu.VMEM((CHROWS, WIN), jnp.int32),
            pltpu.VMEM((CHROWS * WIN, 8), jnp.float32),
            pltpu.VMEM((RPT,), jnp.float32),
            pltpu.VMEM((RPT,), jnp.float32),
            pltpu.VMEM((RPT,), jnp.float32),
            pltpu.VMEM((RPT, 8), jnp.float32),
            pltpu.VMEM((RPT, 8), jnp.float32),
            pltpu.VMEM_SHARED((NP, 8), jnp.float32),
            pltpu.VMEM_SHARED((NP, 8), jnp.float32),
            pltpu.SemaphoreType.DMA,
            pltpu.SemaphoreType.DMA,
            pltpu.SemaphoreType.DMA,
        ],
    )(ei3, xw1, degp, zeros8)


def _glue_layer(g_a0, g_a1, g_prev, g_dis, g_c, g_y, g_t, g_dis8=None,
                scale_prev=False):
    iota = lax.iota(jnp.int32, 16)
    rhalf = lax.shift_right_logical(iota, 3)
    col16 = jnp.bitwise_and(iota, 7)
    rh8 = rhalf * 8
    bv = g_c[0]

    def grp(g, carry):
        rowi = rhalf + 2 * g
        db = plsc.load_gather(g_dis, (rowi,))
        prev = plsc.load_gather(g_prev, (rowi, col16))
        if scale_prev:
            prev = prev * db
        a = (plsc.load_gather(g_a0, (rowi, col16))
             + plsc.load_gather(g_a1, (rowi, col16))
             + prev)
        t = _tanh16(a * db + bv)
        g_t[...] = t
        acc = plsc.load_gather(g_t, (rh8,)) * g_c[1]
        acc = acc + plsc.load_gather(g_t, (rh8 + 1,)) * g_c[2]
        acc = acc + plsc.load_gather(g_t, (rh8 + 2,)) * g_c[3]
        acc = acc + plsc.load_gather(g_t, (rh8 + 3,)) * g_c[4]
        plsc.store_scatter(g_y, (rowi, col16), db * acc)
        if g_dis8 is not None:
            plsc.store_scatter(g_dis8, (rowi, col16), db)
        return carry

    lax.fori_loop(0, GPT, grp, 0, unroll=4)


def _mp2_body(ei_ref, accp_ref, prev_ref, degp_ref, cst_ref, zeros_ref,
              out_ref, ynext_ref,
              isrc_v, idst_v, rows_v, g_d0, g_d1, g_dis,
              g_a0, g_a1, g_prev, g_c, g_t, g_y, y_sh, acc_sh,
              gsem_a, gsem_b, ssem):
    c = lax.axis_index("c")
    s = lax.axis_index("s")
    w = c * NS + s

    @pl.when(s == 0)
    def _():
        pltpu.sync_copy(zeros_ref, acc_sh)

    pltpu.sync_copy(degp_ref.at[0, pl.ds(s * RPT, RPT)], g_d0)
    pltpu.sync_copy(degp_ref.at[1, pl.ds(s * RPT, RPT)], g_d1)
    pltpu.sync_copy(accp_ref.at[0, pl.ds(s * RPT, RPT)], g_a0)
    pltpu.sync_copy(accp_ref.at[1, pl.ds(s * RPT, RPT)], g_a1)
    pltpu.sync_copy(prev_ref.at[pl.ds(s * RPT, RPT)], g_prev)
    pltpu.sync_copy(cst_ref, g_c)
    _fill_dis(g_d0, g_d1, g_dis)
    _glue_layer(g_a0, g_a1, g_prev, g_dis, g_c, g_y, g_t, scale_prev=True)
    pltpu.sync_copy(g_y, y_sh.at[pl.ds(s * RPT, RPT)])

    @pl.when(c == 0)
    def _():
        pltpu.sync_copy(g_y, ynext_ref.at[pl.ds(s * RPT, RPT)])

    plsc.subcore_barrier()
    _edge_pass(w, ei_ref, y_sh, acc_sh, isrc_v, idst_v, rows_v, gsem_a, gsem_b, ssem)
    plsc.subcore_barrier()
    pltpu.sync_copy(acc_sh.at[pl.ds(s * RPT, RPT)],
                    out_ref.at[c, pl.ds(s * RPT, RPT)])


def _mp2_call(ei3, accp, prev_y, degp, cst, zeros8):
    return pl.kernel(
        _mp2_body,
        out_type=[
            jax.ShapeDtypeStruct((NC, NP, 8), jnp.float32),
            jax.ShapeDtypeStruct((NP, 8), jnp.float32),
        ],
        mesh=_mesh,
        compiler_params=_sc_params,
        scratch_types=[
            pltpu.VMEM((CHROWS, WIN), jnp.int32),
            pltpu.VMEM((CHROWS, WIN), jnp.int32),
            pltpu.VMEM((CHROWS * WIN, 8), jnp.float32),
            pltpu.VMEM((RPT,), jnp.float32),
            pltpu.VMEM((RPT,), jnp.float32),
            pltpu.VMEM((RPT,), jnp.float32),
            pltpu.VMEM((RPT, 8), jnp.float32),
            pltpu.VMEM((RPT, 8), jnp.float32),
            pltpu.VMEM((RPT, 8), jnp.float32),
            pltpu.VMEM((5, 16), jnp.float32),
            pltpu.VMEM((16,), jnp.float32),
            pltpu.VMEM((RPT, 8), jnp.float32),
            pltpu.VMEM_SHARED((NP, 8), jnp.float32),
            pltpu.VMEM_SHARED((NP, 8), jnp.float32),
            pltpu.SemaphoreType.DMA,
            pltpu.SemaphoreType.DMA,
            pltpu.SemaphoreType.DMA,
        ],
    )(ei3, accp, prev_y, degp, cst, zeros8)


def _mp3_body(ei_ref, accp_ref, prev_ref, degp_ref, cst_ref, zeros_ref,
              out_ref, ynext_ref, dis8_ref,
              isrc_v, idst_v, rows_v, g_d0, g_d1, g_dis,
              g_a0, g_a1, g_prev, g_c, g_t, g_y, g_dis8, y_sh, acc_sh,
              gsem_a, gsem_b, ssem):
    c = lax.axis_index("c")
    s = lax.axis_index("s")
    w = c * NS + s

    @pl.when(s == 0)
    def _():
        pltpu.sync_copy(zeros_ref, acc_sh)

    pltpu.sync_copy(degp_ref.at[0, pl.ds(s * RPT, RPT)], g_d0)
    pltpu.sync_copy(degp_ref.at[1, pl.ds(s * RPT, RPT)], g_d1)
    pltpu.sync_copy(accp_ref.at[0, pl.ds(s * RPT, RPT)], g_a0)
    pltpu.sync_copy(accp_ref.at[1, pl.ds(s * RPT, RPT)], g_a1)
    pltpu.sync_copy(prev_ref.at[pl.ds(s * RPT, RPT)], g_prev)
    pltpu.sync_copy(cst_ref, g_c)
    _fill_dis(g_d0, g_d1, g_dis)
    _glue_layer(g_a0, g_a1, g_prev, g_dis, g_c, g_y, g_t, g_dis8=g_dis8)
    pltpu.sync_copy(g_y, y_sh.at[pl.ds(s * RPT, RPT)])

    @pl.when(c == 0)
    def _():
        pltpu.sync_copy(g_y, ynext_ref.at[pl.ds(s * RPT, RPT)])
        pltpu.sync_copy(g_dis8, dis8_ref.at[pl.ds(s * RPT, RPT)])

    plsc.subcore_barrier()
    _edge_pass(w, ei_ref, y_sh, acc_sh, isrc_v, idst_v, rows_v, gsem_a, gsem_b, ssem)
    plsc.subcore_barrier()
    pltpu.sync_copy(acc_sh.at[pl.ds(s * RPT, RPT)],
                    out_ref.at[c, pl.ds(s * RPT, RPT)])


def _mp3_call(ei3, accp, prev_y, degp, cst, zeros8):
    return pl.kernel(
        _mp3_body,
        out_type=[
            jax.ShapeDtypeStruct((NC, NP, 8), jnp.float32),
            jax.ShapeDtypeStruct((NP, 8), jnp.float32),
            jax.ShapeDtypeStruct((NP, 8), jnp.float32),
        ],
        mesh=_mesh,
        compiler_params=_sc_params,
        scratch_types=[
            pltpu.VMEM((CHROWS, WIN), jnp.int32),
            pltpu.VMEM((CHROWS, WIN), jnp.int32),
            pltpu.VMEM((CHROWS * WIN, 8), jnp.float32),
            pltpu.VMEM((RPT,), jnp.float32),
            pltpu.VMEM((RPT,), jnp.float32),
            pltpu.VMEM((RPT,), jnp.float32),
            pltpu.VMEM((RPT, 8), jnp.float32),
            pltpu.VMEM((RPT, 8), jnp.float32),
            pltpu.VMEM((RPT, 8), jnp.float32),
            pltpu.VMEM((5, 16), jnp.float32),
            pltpu.VMEM((16,), jnp.float32),
            pltpu.VMEM((RPT, 8), jnp.float32),
            pltpu.VMEM((RPT, 8), jnp.float32),
            pltpu.VMEM_SHARED((NP, 8), jnp.float32),
            pltpu.VMEM_SHARED((NP, 8), jnp.float32),
            pltpu.SemaphoreType.DMA,
            pltpu.SemaphoreType.DMA,
            pltpu.SemaphoreType.DMA,
        ],
    )(ei3, accp, prev_y, degp, cst, zeros8)


def _tca_body(x_ref, w1_ref, y_ref):
    y_ref[...] = jnp.dot(x_ref[...], w1_ref[...],
                         preferred_element_type=jnp.float32)


def _tca(x_pad, w1p):
    return pl.pallas_call(
        _tca_body,
        out_shape=jax.ShapeDtypeStruct((NP, 8), jnp.float32),
    )(x_pad, w1p)


def _tcd_body(accp_ref, y_ref, dis8_ref, b_ref, wc_ref, bc_ref, out_ref, h_ref):
    dis8 = dis8_ref[...]
    a = (accp_ref[0] + accp_ref[1] + y_ref[...]) * dis8 + b_ref[...]
    h = jnp.tanh(a)
    h_ref[...] = h[:, 0:2]
    out_ref[...] = (h[:, 0:1] * wc_ref[0:1, :] + h[:, 1:2] * wc_ref[1:2, :]
                    + bc_ref[...])


def _tcd(accp, y, dis8, b2d, wc, bc2d):
    return pl.pallas_call(
        _tcd_body,
        out_shape=[
            jax.ShapeDtypeStruct((NP, 4), jnp.float32),
            jax.ShapeDtypeStruct((NP, 2), jnp.float32),
        ],
    )(accp, y, dis8, b2d, wc, bc2d)


def _const_block(b, w):
    bp = jnp.pad(b, (0, 8 - b.shape[0]))
    wp = jnp.pad(w, ((0, 4 - w.shape[0]), (0, 8 - w.shape[1])))
    rows = [jnp.tile(bp, 2)] + [jnp.tile(wp[k], 2) for k in range(4)]
    return jnp.stack(rows)


def kernel(x, edge_index, W1, b1, W2, b2, W3, b3, Wc, bc):
    ei3 = edge_index.astype(jnp.int32).reshape(2, NWIN, WIN)
    x_pad = jnp.pad(x, ((0, NP - N), (0, 0)))
    w1p = jnp.pad(W1, ((0, 0), (0, 4)))
    zeros1 = jnp.zeros((NP,), jnp.float32)
    zeros8 = jnp.zeros((NP, 8), jnp.float32)
    cst1 = _const_block(b1, W2)
    cst2 = _const_block(b2, W3)

    xw1 = _tca(x_pad, w1p)
    degp = _deg_call(ei3, zeros1)

    acc1 = _mp1_call(ei3, xw1, degp, zeros8)
    acc2, y2 = _mp2_call(ei3, acc1, xw1, degp, cst1, zeros8)
    acc3, y3, dis8 = _mp3_call(ei3, acc2, y2, degp, cst2, zeros8)

    out_full, h_full = _tcd(acc3, y3, dis8, jnp.pad(b3, (0, 6)).reshape(1, 8),
                            Wc, bc.reshape(1, 4))
    return (out_full[:N], h_full[:N])


# speedup vs baseline: 63.1404x; 1.0332x over previous
import functools

import jax
import jax.numpy as jnp
from jax import lax
from jax.experimental import pallas as pl
from jax.experimental.pallas import tpu as pltpu
from jax.experimental.pallas import tpu_sc as plsc

N = 10000
E = 320000
NC = 2
NS = 16
NP = 10240
RPT = NP // NS
GPT = RPT // 2
WIN = 128
NWIN = E // WIN
WPW = 78
CHROWS = 13
NCHUNK = WPW // CHROWS
NEXTRA = NWIN - 32 * WPW

_mesh = plsc.VectorSubcoreMesh(
    core_axis_name="c", subcore_axis_name="s", num_cores=NC, num_subcores=NS
)
_sc_params = pltpu.CompilerParams(use_tc_tiling_on_sc=False,
                                  needs_layout_passes=False)


def _tanh16(v):
    av = jnp.abs(v)
    e = jnp.exp(av * -2.0)
    t = (1.0 - e) / (1.0 + e)
    return jnp.where(v < 0.0, -t, t)


def _fill_dis(g_d0, g_d1, g_dis):
    def body(j, carry):
        d = g_d0[pl.ds(16 * j, 16)] + g_d1[pl.ds(16 * j, 16)] + 1.0
        i = plsc.bitcast(d, jnp.int32)
        i = 0x5F3759DF - lax.shift_right_logical(i, 1)
        r = plsc.bitcast(i, jnp.float32)
        r = r * (1.5 - 0.5 * d * r * r)
        r = r * (1.5 - 0.5 * d * r * r)
        r = r * (1.5 - 0.5 * d * r * r)
        g_dis[pl.ds(16 * j, 16)] = r
        return carry

    lax.fori_loop(0, RPT // 16, body, 0, unroll=2)


def _edge_pass(w, ei_ref, y_sh, acc_sh, isrc_v, idst_v, rows_v,
               gsem_a, gsem_b, ssem):
    gsems = (gsem_a, gsem_b)

    def chunk(i, carry):
        r0 = w * WPW + i * CHROWS
        pltpu.sync_copy(ei_ref.at[0, pl.ds(r0, CHROWS)], isrc_v)
        pltpu.sync_copy(ei_ref.at[1, pl.ds(r0, CHROWS)], idst_v)
        gath = [None] * CHROWS
        gath[0] = pltpu.async_copy(y_sh.at[isrc_v.at[0]],
                                   rows_v.at[pl.ds(0, WIN)], gsems[0])
        scat = []
        for j in range(CHROWS):
            if j + 1 < CHROWS:
                gath[j + 1] = pltpu.async_copy(
                    y_sh.at[isrc_v.at[j + 1]],
                    rows_v.at[pl.ds((j + 1) * WIN, WIN)], gsems[(j + 1) % 2])
            gath[j].wait()
            scat.append(pltpu.async_copy(rows_v.at[pl.ds(j * WIN, WIN)],
                                         acc_sh.at[idst_v.at[j]], ssem,
                                         add=True))
        for d in scat:
            d.wait()
        return carry

    lax.fori_loop(0, NCHUNK, chunk, 0)

    @pl.when(w < NEXTRA)
    def _():
        r0 = 32 * WPW + w
        pltpu.sync_copy(ei_ref.at[0, pl.ds(r0, 1)], isrc_v.at[pl.ds(0, 1)])
        pltpu.sync_copy(ei_ref.at[1, pl.ds(r0, 1)], idst_v.at[pl.ds(0, 1)])
        pltpu.sync_copy(y_sh.at[isrc_v.at[0]], rows_v.at[pl.ds(0, WIN)])
        pltpu.sync_copy(rows_v.at[pl.ds(0, WIN)], acc_sh.at[idst_v.at[0]],
                        add=True)


def _deg_body(ei_ref, zeros_ref, out_ref, idx_v, ones_v, deg_sh):
    c = lax.axis_index("c")
    s = lax.axis_index("s")
    w = c * NS + s

    @pl.when(s == 0)
    def _():
        pltpu.sync_copy(zeros_ref, deg_sh)

    for k in range(WIN // 16):
        ones_v[pl.ds(k * 16, 16)] = jnp.ones((16,), jnp.float32)
    plsc.subcore_barrier()

    def chunk(i, carry):
        r0 = w * WPW + i * CHROWS
        pltpu.sync_copy(ei_ref.at[1, pl.ds(r0, CHROWS)], idx_v)
        for j in range(CHROWS):
            pltpu.sync_copy(ones_v, deg_sh.at[idx_v.at[j]], add=True)
        return carry

    lax.fori_loop(0, NCHUNK, chunk, 0)

    @pl.when(w < NEXTRA)
    def _():
        r0 = 32 * WPW + w
        pltpu.sync_copy(ei_ref.at[1, pl.ds(r0, 1)], idx_v.at[pl.ds(0, 1)])
        pltpu.sync_copy(ones_v, deg_sh.at[idx_v.at[0]], add=True)

    plsc.subcore_barrier()
    pltpu.sync_copy(deg_sh.at[pl.ds(s * RPT, RPT)],
                    out_ref.at[c, pl.ds(s * RPT, RPT)])


def _deg_call(ei3, zeros1):
    return pl.kernel(
        _deg_body,
        out_type=jax.ShapeDtypeStruct((NC, NP), jnp.float32),
        mesh=_mesh,
        compiler_params=_sc_params,
        scratch_types=[
            pltpu.VMEM((CHROWS, WIN), jnp.int32),
            pltpu.VMEM((WIN,), jnp.float32),
            pltpu.VMEM_SHARED((NP,), jnp.float32),
        ],
    )(ei3, zeros1)


def _mp1_body(ei_ref, xw_ref, degp_ref, zeros_ref, out_ref,
              isrc_v, idst_v, rows_v, g_d0, g_d1, g_dis, g_in, g_y,
              y_sh, acc_sh, gsem_a, gsem_b, ssem):
    c = lax.axis_index("c")
    s = lax.axis_index("s")
    w = c * NS + s

    @pl.when(s == 0)
    def _():
        pltpu.sync_copy(zeros_ref, acc_sh)

    pltpu.sync_copy(degp_ref.at[0, pl.ds(s * RPT, RPT)], g_d0)
    pltpu.sync_copy(degp_ref.at[1, pl.ds(s * RPT, RPT)], g_d1)
    pltpu.sync_copy(xw_ref.at[pl.ds(s * RPT, RPT)], g_in)
    _fill_dis(g_d0, g_d1, g_dis)

    iota = lax.iota(jnp.int32, 16)
    rhalf = lax.shift_right_logical(iota, 3)
    col16 = jnp.bitwise_and(iota, 7)

    def grp(g, carry):
        rowi = rhalf + 2 * g
        db = plsc.load_gather(g_dis, (rowi,))
        xw = plsc.load_gather(g_in, (rowi, col16))
        plsc.store_scatter(g_y, (rowi, col16), db * xw)
        return carry

    lax.fori_loop(0, GPT, grp, 0, unroll=4)
    pltpu.sync_copy(g_y, y_sh.at[pl.ds(s * RPT, RPT)])
    plsc.subcore_barrier()
    _edge_pass(w, ei_ref, y_sh, acc_sh, isrc_v, idst_v, rows_v, gsem_a, gsem_b, ssem)
    plsc.subcore_barrier()
    pltpu.sync_copy(acc_sh.at[pl.ds(s * RPT, RPT)],
                    out_ref.at[c, pl.ds(s * RPT, RPT)])


def _mp1_call(ei3, xw1, degp, zeros8):
    return pl.kernel(
        _mp1_body,
        out_type=jax.ShapeDtypeStruct((NC, NP, 8), jnp.float32),
        mesh=_mesh,
        compiler_params=_sc_params,
        scratch_types=[
            pltpu.VMEM((CHROWS, WIN), jnp.int32),
            pltpu.VMEM((CHROWS, WIN), jnp.int32),
            pltpu.VMEM((CHROWS * WIN, 8), jnp.float32),
            pltpu.VMEM((RPT,), jnp.float32),
            pltpu.VMEM((RPT,), jnp.float32),
            pltpu.VMEM((RPT,), jnp.float32),
            pltpu.VMEM((RPT, 8), jnp.float32),
            pltpu.VMEM((RPT, 8), jnp.float32),
            pltpu.VMEM_SHARED((NP, 8), jnp.float32),
            pltpu.VMEM_SHARED((NP, 8), jnp.float32),
            pltpu.SemaphoreType.DMA,
            pltpu.SemaphoreType.DMA,
            pltpu.SemaphoreType.DMA,
        ],
    )(ei3, xw1, degp, zeros8)


def _glue_layer(g_a0, g_a1, g_prev, g_dis, g_c, g_y, g_t, g_dis8=None,
                scale_prev=False):
    iota = lax.iota(jnp.int32, 16)
    rhalf = lax.shift_right_logical(iota, 3)
    col16 = jnp.bitwise_and(iota, 7)
    rh8 = rhalf * 8
    bv = g_c[0]

    def grp(g, carry):
        rowi = rhalf + 2 * g
        db = plsc.load_gather(g_dis, (rowi,))
        prev = plsc.load_gather(g_prev, (rowi, col16))
        if scale_prev:
            prev = prev * db
        a = (plsc.load_gather(g_a0, (rowi, col16))
             + plsc.load_gather(g_a1, (rowi, col16))
             + prev)
        t = _tanh16(a * db + bv)
        g_t[...] = t
        acc = plsc.load_gather(g_t, (rh8,)) * g_c[1]
        acc = acc + plsc.load_gather(g_t, (rh8 + 1,)) * g_c[2]
        acc = acc + plsc.load_gather(g_t, (rh8 + 2,)) * g_c[3]
        acc = acc + plsc.load_gather(g_t, (rh8 + 3,)) * g_c[4]
        plsc.store_scatter(g_y, (rowi, col16), db * acc)
        if g_dis8 is not None:
            plsc.store_scatter(g_dis8, (rowi, col16), db)
        return carry

    lax.fori_loop(0, GPT, grp, 0, unroll=4)


def _mp2_body(scale_prev, ei_ref, accp_ref, prev_ref, degp_ref, cst_ref, zeros_ref,
              out_ref, ynext_ref,
              isrc_v, idst_v, rows_v, g_d0, g_d1, g_dis,
              g_a0, g_a1, g_prev, g_c, g_t, g_y, y_sh, acc_sh,
              gsem_a, gsem_b, ssem):
    c = lax.axis_index("c")
    s = lax.axis_index("s")
    w = c * NS + s

    @pl.when(s == 0)
    def _():
        pltpu.sync_copy(zeros_ref, acc_sh)

    pltpu.sync_copy(degp_ref.at[0, pl.ds(s * RPT, RPT)], g_d0)
    pltpu.sync_copy(degp_ref.at[1, pl.ds(s * RPT, RPT)], g_d1)
    pltpu.sync_copy(accp_ref.at[0, pl.ds(s * RPT, RPT)], g_a0)
    pltpu.sync_copy(accp_ref.at[1, pl.ds(s * RPT, RPT)], g_a1)
    pltpu.sync_copy(prev_ref.at[pl.ds(s * RPT, RPT)], g_prev)
    pltpu.sync_copy(cst_ref, g_c)
    _fill_dis(g_d0, g_d1, g_dis)
    _glue_layer(g_a0, g_a1, g_prev, g_dis, g_c, g_y, g_t, scale_prev=scale_prev)
    pltpu.sync_copy(g_y, y_sh.at[pl.ds(s * RPT, RPT)])

    @pl.when(c == 0)
    def _():
        pltpu.sync_copy(g_y, ynext_ref.at[pl.ds(s * RPT, RPT)])

    plsc.subcore_barrier()
    _edge_pass(w, ei_ref, y_sh, acc_sh, isrc_v, idst_v, rows_v, gsem_a, gsem_b, ssem)
    plsc.subcore_barrier()
    pltpu.sync_copy(acc_sh.at[pl.ds(s * RPT, RPT)],
                    out_ref.at[c, pl.ds(s * RPT, RPT)])


def _mp2_call(ei3, accp, prev_y, degp, cst, zeros8, scale_prev=True):
    return pl.kernel(
        functools.partial(_mp2_body, scale_prev),
        out_type=[
            jax.ShapeDtypeStruct((NC, NP, 8), jnp.float32),
            jax.ShapeDtypeStruct((NP, 8), jnp.float32),
        ],
        mesh=_mesh,
        compiler_params=_sc_params,
        scratch_types=[
            pltpu.VMEM((CHROWS, WIN), jnp.int32),
            pltpu.VMEM((CHROWS, WIN), jnp.int32),
            pltpu.VMEM((CHROWS * WIN, 8), jnp.float32),
            pltpu.VMEM((RPT,), jnp.float32),
            pltpu.VMEM((RPT,), jnp.float32),
            pltpu.VMEM((RPT,), jnp.float32),
            pltpu.VMEM((RPT, 8), jnp.float32),
            pltpu.VMEM((RPT, 8), jnp.float32),
            pltpu.VMEM((RPT, 8), jnp.float32),
            pltpu.VMEM((5, 16), jnp.float32),
            pltpu.VMEM((16,), jnp.float32),
            pltpu.VMEM((RPT, 8), jnp.float32),
            pltpu.VMEM_SHARED((NP, 8), jnp.float32),
            pltpu.VMEM_SHARED((NP, 8), jnp.float32),
            pltpu.SemaphoreType.DMA,
            pltpu.SemaphoreType.DMA,
            pltpu.SemaphoreType.DMA,
        ],
    )(ei3, accp, prev_y, degp, cst, zeros8)


def _fin_body(accp_ref, prev_ref, degp_ref, cst_ref, out8_ref, h8_ref,
              g_d0, g_d1, g_dis, g_a0, g_a1, g_prev, g_c, g_t, g_o, g_h):
    c = lax.axis_index("c")
    s = lax.axis_index("s")
    w = c * NS + s
    rpw = NP // 32
    r0 = w * rpw

    pltpu.sync_copy(degp_ref.at[0, pl.ds(r0, rpw)], g_d0)
    pltpu.sync_copy(degp_ref.at[1, pl.ds(r0, rpw)], g_d1)
    pltpu.sync_copy(accp_ref.at[0, pl.ds(r0, rpw)], g_a0)
    pltpu.sync_copy(accp_ref.at[1, pl.ds(r0, rpw)], g_a1)
    pltpu.sync_copy(prev_ref.at[pl.ds(r0, rpw)], g_prev)
    pltpu.sync_copy(cst_ref, g_c)

    def fill(j, carry):
        d = g_d0[pl.ds(16 * j, 16)] + g_d1[pl.ds(16 * j, 16)] + 1.0
        i = plsc.bitcast(d, jnp.int32)
        i = 0x5F3759DF - lax.shift_right_logical(i, 1)
        r = plsc.bitcast(i, jnp.float32)
        r = r * (1.5 - 0.5 * d * r * r)
        r = r * (1.5 - 0.5 * d * r * r)
        r = r * (1.5 - 0.5 * d * r * r)
        g_dis[pl.ds(16 * j, 16)] = r
        return carry

    lax.fori_loop(0, rpw // 16, fill, 0, unroll=2)

    iota = lax.iota(jnp.int32, 16)
    rhalf = lax.shift_right_logical(iota, 3)
    col16 = jnp.bitwise_and(iota, 7)
    rh8 = rhalf * 8
    bv = g_c[0]
    bcv = g_c[3]

    def grp(g, carry):
        rowi = rhalf + 2 * g
        db = plsc.load_gather(g_dis, (rowi,))
        a = (plsc.load_gather(g_a0, (rowi, col16))
             + plsc.load_gather(g_a1, (rowi, col16))
             + plsc.load_gather(g_prev, (rowi, col16)))
        t = _tanh16(a * db + bv)
        g_t[...] = t
        o = (plsc.load_gather(g_t, (rh8,)) * g_c[1]
             + plsc.load_gather(g_t, (rh8 + 1,)) * g_c[2] + bcv)
        plsc.store_scatter(g_o, (rowi, col16), o)
        plsc.store_scatter(g_h, (rowi, col16), t)
        return carry

    lax.fori_loop(0, rpw // 2, grp, 0, unroll=4)
    pltpu.sync_copy(g_o, out8_ref.at[pl.ds(r0, rpw)])
    pltpu.sync_copy(g_h, h8_ref.at[pl.ds(r0, rpw)])


def _fin_call(accp, y3, degp, cstf):
    rpw = NP // 32
    return pl.kernel(
        _fin_body,
        out_type=[
            jax.ShapeDtypeStruct((NP, 8), jnp.float32),
            jax.ShapeDtypeStruct((NP, 8), jnp.float32),
        ],
        mesh=_mesh,
        compiler_params=_sc_params,
        scratch_types=[
            pltpu.VMEM((rpw,), jnp.float32),
            pltpu.VMEM((rpw,), jnp.float32),
            pltpu.VMEM((rpw,), jnp.float32),
            pltpu.VMEM((rpw, 8), jnp.float32),
            pltpu.VMEM((rpw, 8), jnp.float32),
            pltpu.VMEM((rpw, 8), jnp.float32),
            pltpu.VMEM((4, 16), jnp.float32),
            pltpu.VMEM((16,), jnp.float32),
            pltpu.VMEM((rpw, 8), jnp.float32),
            pltpu.VMEM((rpw, 8), jnp.float32),
        ],
    )(accp, y3, degp, cstf)


def _tca_body(x_ref, w1_ref, y_ref):
    y_ref[...] = jnp.dot(x_ref[...], w1_ref[...],
                         preferred_element_type=jnp.float32)


def _tca(x_pad, w1p):
    return pl.pallas_call(
        _tca_body,
        out_shape=jax.ShapeDtypeStruct((NP, 8), jnp.float32),
    )(x_pad, w1p)


def _const_block(b, w):
    bp = jnp.pad(b, (0, 8 - b.shape[0]))
    wp = jnp.pad(w, ((0, 4 - w.shape[0]), (0, 8 - w.shape[1])))
    rows = [jnp.tile(bp, 2)] + [jnp.tile(wp[k], 2) for k in range(4)]
    return jnp.stack(rows)


def kernel(x, edge_index, W1, b1, W2, b2, W3, b3, Wc, bc):
    ei3 = edge_index.astype(jnp.int32).reshape(2, NWIN, WIN)
    x_pad = jnp.pad(x, ((0, NP - N), (0, 0)))
    w1p = jnp.pad(W1, ((0, 0), (0, 4)))
    zeros1 = jnp.zeros((NP,), jnp.float32)
    zeros8 = jnp.zeros((NP, 8), jnp.float32)
    cst1 = _const_block(b1, W2)
    cst2 = _const_block(b2, W3)

    cstf = jnp.stack([
        jnp.tile(jnp.pad(b3, (0, 6)), 2),
        jnp.tile(jnp.pad(Wc[0], (0, 4)), 2),
        jnp.tile(jnp.pad(Wc[1], (0, 4)), 2),
        jnp.tile(jnp.pad(bc, (0, 4)), 2),
    ])

    xw1 = _tca(x_pad, w1p)
    degp = _deg_call(ei3, zeros1)

    acc1 = _mp1_call(ei3, xw1, degp, zeros8)
    acc2, y2 = _mp2_call(ei3, acc1, xw1, degp, cst1, zeros8)
    acc3, y3 = _mp2_call(ei3, acc2, y2, degp, cst2, zeros8, scale_prev=False)

    out8, h8 = _fin_call(acc3, y3, degp, cstf)
    return (out8[:N, :4], h8[:N, :2])


# speedup vs baseline: 65.1126x; 1.0312x over previous
import functools

import jax
import jax.numpy as jnp
from jax import lax
from jax.experimental import pallas as pl
from jax.experimental.pallas import tpu as pltpu
from jax.experimental.pallas import tpu_sc as plsc

N = 10000
E = 320000
NC = 2
NS = 16
NP = 10240
RPT = NP // NS
GPT = RPT // 2
WIN = 128
NWIN = E // WIN
WPW = 78
CHROWS = 13
NCHUNK = WPW // CHROWS
NEXTRA = NWIN - 32 * WPW

_mesh = plsc.VectorSubcoreMesh(
    core_axis_name="c", subcore_axis_name="s", num_cores=NC, num_subcores=NS
)
_sc_params = pltpu.CompilerParams(use_tc_tiling_on_sc=False,
                                  needs_layout_passes=False)


def _tanh16(v):
    av = jnp.abs(v)
    e = jnp.exp(av * -2.0)
    t = (1.0 - e) / (1.0 + e)
    return jnp.where(v < 0.0, -t, t)


def _fill_dis(g_d0, g_d1, g_dis):
    def body(j, carry):
        d = g_d0[pl.ds(16 * j, 16)] + g_d1[pl.ds(16 * j, 16)] + 1.0
        i = plsc.bitcast(d, jnp.int32)
        i = 0x5F3759DF - lax.shift_right_logical(i, 1)
        r = plsc.bitcast(i, jnp.float32)
        r = r * (1.5 - 0.5 * d * r * r)
        r = r * (1.5 - 0.5 * d * r * r)
        r = r * (1.5 - 0.5 * d * r * r)
        g_dis[pl.ds(16 * j, 16)] = r
        return carry

    lax.fori_loop(0, RPT // 16, body, 0, unroll=2)


def _edge_pass(w, ei_ref, y_sh, acc_sh, isrc_v, idst_v, rows_v,
               gsem_a, gsem_b, ssem):
    gsems = (gsem_a, gsem_b)

    def chunk(i, carry):
        r0 = w * WPW + i * CHROWS
        pltpu.sync_copy(ei_ref.at[0, pl.ds(r0, CHROWS)], isrc_v)
        pltpu.sync_copy(ei_ref.at[1, pl.ds(r0, CHROWS)], idst_v)
        gath = [None] * CHROWS
        gath[0] = pltpu.async_copy(y_sh.at[isrc_v.at[0]],
                                   rows_v.at[pl.ds(0, WIN)], gsems[0])
        scat = []
        for j in range(CHROWS):
            if j + 1 < CHROWS:
                gath[j + 1] = pltpu.async_copy(
                    y_sh.at[isrc_v.at[j + 1]],
                    rows_v.at[pl.ds((j + 1) * WIN, WIN)], gsems[(j + 1) % 2])
            gath[j].wait()
            scat.append(pltpu.async_copy(rows_v.at[pl.ds(j * WIN, WIN)],
                                         acc_sh.at[idst_v.at[j]], ssem,
                                         add=True))
        for d in scat:
            d.wait()
        return carry

    lax.fori_loop(0, NCHUNK, chunk, 0)

    @pl.when(w < NEXTRA)
    def _():
        r0 = 32 * WPW + w
        pltpu.sync_copy(ei_ref.at[0, pl.ds(r0, 1)], isrc_v.at[pl.ds(0, 1)])
        pltpu.sync_copy(ei_ref.at[1, pl.ds(r0, 1)], idst_v.at[pl.ds(0, 1)])
        pltpu.sync_copy(y_sh.at[isrc_v.at[0]], rows_v.at[pl.ds(0, WIN)])
        pltpu.sync_copy(rows_v.at[pl.ds(0, WIN)], acc_sh.at[idst_v.at[0]],
                        add=True)


def _deg_body(ei_ref, zeros_ref, out_ref, idx_v, ones_v, deg_sh, dsem):
    c = lax.axis_index("c")
    s = lax.axis_index("s")
    w = c * NS + s

    @pl.when(s == 0)
    def _():
        pltpu.sync_copy(zeros_ref, deg_sh)

    for k in range(WIN // 16):
        ones_v[pl.ds(k * 16, 16)] = jnp.ones((16,), jnp.float32)
    plsc.subcore_barrier()

    def chunk(i, carry):
        r0 = w * WPW + i * CHROWS
        pltpu.sync_copy(ei_ref.at[1, pl.ds(r0, CHROWS)], idx_v)
        scat = [pltpu.async_copy(ones_v, deg_sh.at[idx_v.at[j]], dsem,
                                 add=True)
                for j in range(CHROWS)]
        for d in scat:
            d.wait()
        return carry

    lax.fori_loop(0, NCHUNK, chunk, 0)

    @pl.when(w < NEXTRA)
    def _():
        r0 = 32 * WPW + w
        pltpu.sync_copy(ei_ref.at[1, pl.ds(r0, 1)], idx_v.at[pl.ds(0, 1)])
        pltpu.sync_copy(ones_v, deg_sh.at[idx_v.at[0]], add=True)

    plsc.subcore_barrier()
    pltpu.sync_copy(deg_sh.at[pl.ds(s * RPT, RPT)],
                    out_ref.at[c, pl.ds(s * RPT, RPT)])


def _deg_call(ei3, zeros1):
    return pl.kernel(
        _deg_body,
        out_type=jax.ShapeDtypeStruct((NC, NP), jnp.float32),
        mesh=_mesh,
        compiler_params=_sc_params,
        scratch_types=[
            pltpu.VMEM((CHROWS, WIN), jnp.int32),
            pltpu.VMEM((WIN,), jnp.float32),
            pltpu.VMEM_SHARED((NP,), jnp.float32),
            pltpu.SemaphoreType.DMA,
        ],
    )(ei3, zeros1)


def _mp1_body(ei_ref, xw_ref, degp_ref, zeros_ref, out_ref,
              isrc_v, idst_v, rows_v, g_d0, g_d1, g_dis, g_in, g_y,
              y_sh, acc_sh, gsem_a, gsem_b, ssem):
    c = lax.axis_index("c")
    s = lax.axis_index("s")
    w = c * NS + s

    @pl.when(s == 0)
    def _():
        pltpu.sync_copy(zeros_ref, acc_sh)

    pltpu.sync_copy(degp_ref.at[0, pl.ds(s * RPT, RPT)], g_d0)
    pltpu.sync_copy(degp_ref.at[1, pl.ds(s * RPT, RPT)], g_d1)
    pltpu.sync_copy(xw_ref.at[pl.ds(s * RPT, RPT)], g_in)
    _fill_dis(g_d0, g_d1, g_dis)

    iota = lax.iota(jnp.int32, 16)
    rhalf = lax.shift_right_logical(iota, 3)
    col16 = jnp.bitwise_and(iota, 7)

    def grp(g, carry):
        rowi = rhalf + 2 * g
        db = plsc.load_gather(g_dis, (rowi,))
        xw = plsc.load_gather(g_in, (rowi, col16))
        plsc.store_scatter(g_y, (rowi, col16), db * xw)
        return carry

    lax.fori_loop(0, GPT, grp, 0, unroll=4)
    pltpu.sync_copy(g_y, y_sh.at[pl.ds(s * RPT, RPT)])
    plsc.subcore_barrier()
    _edge_pass(w, ei_ref, y_sh, acc_sh, isrc_v, idst_v, rows_v, gsem_a, gsem_b, ssem)
    plsc.subcore_barrier()
    pltpu.sync_copy(acc_sh.at[pl.ds(s * RPT, RPT)],
                    out_ref.at[c, pl.ds(s * RPT, RPT)])


def _mp1_call(ei3, xw1, degp, zeros8):
    return pl.kernel(
        _mp1_body,
        out_type=jax.ShapeDtypeStruct((NC, NP, 8), jnp.float32),
        mesh=_mesh,
        compiler_params=_sc_params,
        scratch_types=[
            pltpu.VMEM((CHROWS, WIN), jnp.int32),
            pltpu.VMEM((CHROWS, WIN), jnp.int32),
            pltpu.VMEM((CHROWS * WIN, 8), jnp.float32),
            pltpu.VMEM((RPT,), jnp.float32),
            pltpu.VMEM((RPT,), jnp.float32),
            pltpu.VMEM((RPT,), jnp.float32),
            pltpu.VMEM((RPT, 8), jnp.float32),
            pltpu.VMEM((RPT, 8), jnp.float32),
            pltpu.VMEM_SHARED((NP, 8), jnp.float32),
            pltpu.VMEM_SHARED((NP, 8), jnp.float32),
            pltpu.SemaphoreType.DMA,
            pltpu.SemaphoreType.DMA,
            pltpu.SemaphoreType.DMA,
        ],
    )(ei3, xw1, degp, zeros8)


def _glue_layer(g_a0, g_a1, g_prev, g_dis, g_c, g_y, g_t, g_dis8=None,
                scale_prev=False):
    iota = lax.iota(jnp.int32, 16)
    rhalf = lax.shift_right_logical(iota, 3)
    col16 = jnp.bitwise_and(iota, 7)
    rh8 = rhalf * 8
    bv = g_c[0]

    def grp(g, carry):
        rowi = rhalf + 2 * g
        db = plsc.load_gather(g_dis, (rowi,))
        prev = plsc.load_gather(g_prev, (rowi, col16))
        if scale_prev:
            prev = prev * db
        a = (plsc.load_gather(g_a0, (rowi, col16))
             + plsc.load_gather(g_a1, (rowi, col16))
             + prev)
        t = _tanh16(a * db + bv)
        acc = t.at[rh8].get(mode="promise_in_bounds") * g_c[1]
        acc = acc + t.at[rh8 + 1].get(mode="promise_in_bounds") * g_c[2]
        acc = acc + t.at[rh8 + 2].get(mode="promise_in_bounds") * g_c[3]
        acc = acc + t.at[rh8 + 3].get(mode="promise_in_bounds") * g_c[4]
        plsc.store_scatter(g_y, (rowi, col16), db * acc)
        if g_dis8 is not None:
            plsc.store_scatter(g_dis8, (rowi, col16), db)
        return carry

    lax.fori_loop(0, GPT, grp, 0, unroll=4)


def _mp2_body(scale_prev, ei_ref, accp_ref, prev_ref, degp_ref, cst_ref, zeros_ref,
              out_ref, ynext_ref,
              isrc_v, idst_v, rows_v, g_d0, g_d1, g_dis,
              g_a0, g_a1, g_prev, g_c, g_t, g_y, y_sh, acc_sh,
              gsem_a, gsem_b, ssem):
    c = lax.axis_index("c")
    s = lax.axis_index("s")
    w = c * NS + s

    @pl.when(s == 0)
    def _():
        pltpu.sync_copy(zeros_ref, acc_sh)

    pltpu.sync_copy(degp_ref.at[0, pl.ds(s * RPT, RPT)], g_d0)
    pltpu.sync_copy(degp_ref.at[1, pl.ds(s * RPT, RPT)], g_d1)
    pltpu.sync_copy(accp_ref.at[0, pl.ds(s * RPT, RPT)], g_a0)
    pltpu.sync_copy(accp_ref.at[1, pl.ds(s * RPT, RPT)], g_a1)
    pltpu.sync_copy(prev_ref.at[pl.ds(s * RPT, RPT)], g_prev)
    pltpu.sync_copy(cst_ref, g_c)
    _fill_dis(g_d0, g_d1, g_dis)
    _glue_layer(g_a0, g_a1, g_prev, g_dis, g_c, g_y, g_t, scale_prev=scale_prev)
    pltpu.sync_copy(g_y, y_sh.at[pl.ds(s * RPT, RPT)])

    @pl.when(c == 0)
    def _():
        pltpu.sync_copy(g_y, ynext_ref.at[pl.ds(s * RPT, RPT)])

    plsc.subcore_barrier()
    _edge_pass(w, ei_ref, y_sh, acc_sh, isrc_v, idst_v, rows_v, gsem_a, gsem_b, ssem)
    plsc.subcore_barrier()
    pltpu.sync_copy(acc_sh.at[pl.ds(s * RPT, RPT)],
                    out_ref.at[c, pl.ds(s * RPT, RPT)])


def _mp2_call(ei3, accp, prev_y, degp, cst, zeros8, scale_prev=True):
    return pl.kernel(
        functools.partial(_mp2_body, scale_prev),
        out_type=[
            jax.ShapeDtypeStruct((NC, NP, 8), jnp.float32),
            jax.ShapeDtypeStruct((NP, 8), jnp.float32),
        ],
        mesh=_mesh,
        compiler_params=_sc_params,
        scratch_types=[
            pltpu.VMEM((CHROWS, WIN), jnp.int32),
            pltpu.VMEM((CHROWS, WIN), jnp.int32),
            pltpu.VMEM((CHROWS * WIN, 8), jnp.float32),
            pltpu.VMEM((RPT,), jnp.float32),
            pltpu.VMEM((RPT,), jnp.float32),
            pltpu.VMEM((RPT,), jnp.float32),
            pltpu.VMEM((RPT, 8), jnp.float32),
            pltpu.VMEM((RPT, 8), jnp.float32),
            pltpu.VMEM((RPT, 8), jnp.float32),
            pltpu.VMEM((5, 16), jnp.float32),
            pltpu.VMEM((16,), jnp.float32),
            pltpu.VMEM((RPT, 8), jnp.float32),
            pltpu.VMEM_SHARED((NP, 8), jnp.float32),
            pltpu.VMEM_SHARED((NP, 8), jnp.float32),
            pltpu.SemaphoreType.DMA,
            pltpu.SemaphoreType.DMA,
            pltpu.SemaphoreType.DMA,
        ],
    )(ei3, accp, prev_y, degp, cst, zeros8)


def _fin_body(accp_ref, prev_ref, degp_ref, cst_ref, out8_ref, h8_ref,
              g_d0, g_d1, g_dis, g_a0, g_a1, g_prev, g_c, g_t, g_o, g_h):
    c = lax.axis_index("c")
    s = lax.axis_index("s")
    w = c * NS + s
    rpw = NP // 32
    r0 = w * rpw

    pltpu.sync_copy(degp_ref.at[0, pl.ds(r0, rpw)], g_d0)
    pltpu.sync_copy(degp_ref.at[1, pl.ds(r0, rpw)], g_d1)
    pltpu.sync_copy(accp_ref.at[0, pl.ds(r0, rpw)], g_a0)
    pltpu.sync_copy(accp_ref.at[1, pl.ds(r0, rpw)], g_a1)
    pltpu.sync_copy(prev_ref.at[pl.ds(r0, rpw)], g_prev)
    pltpu.sync_copy(cst_ref, g_c)

    def fill(j, carry):
        d = g_d0[pl.ds(16 * j, 16)] + g_d1[pl.ds(16 * j, 16)] + 1.0
        i = plsc.bitcast(d, jnp.int32)
        i = 0x5F3759DF - lax.shift_right_logical(i, 1)
        r = plsc.bitcast(i, jnp.float32)
        r = r * (1.5 - 0.5 * d * r * r)
        r = r * (1.5 - 0.5 * d * r * r)
        r = r * (1.5 - 0.5 * d * r * r)
        g_dis[pl.ds(16 * j, 16)] = r
        return carry

    lax.fori_loop(0, rpw // 16, fill, 0, unroll=2)

    iota = lax.iota(jnp.int32, 16)
    rhalf = lax.shift_right_logical(iota, 3)
    col16 = jnp.bitwise_and(iota, 7)
    rh8 = rhalf * 8
    bv = g_c[0]
    bcv = g_c[3]

    def grp(g, carry):
        rowi = rhalf + 2 * g
        db = plsc.load_gather(g_dis, (rowi,))
        a = (plsc.load_gather(g_a0, (rowi, col16))
             + plsc.load_gather(g_a1, (rowi, col16))
             + plsc.load_gather(g_prev, (rowi, col16)))
        t = _tanh16(a * db + bv)
        o = (t.at[rh8].get(mode="promise_in_bounds") * g_c[1]
             + t.at[rh8 + 1].get(mode="promise_in_bounds") * g_c[2] + bcv)
        plsc.store_scatter(g_o, (rowi, col16), o)
        plsc.store_scatter(g_h, (rowi, col16), t)
        return carry

    lax.fori_loop(0, rpw // 2, grp, 0, unroll=4)
    pltpu.sync_copy(g_o, out8_ref.at[pl.ds(r0, rpw)])
    pltpu.sync_copy(g_h, h8_ref.at[pl.ds(r0, rpw)])


def _fin_call(accp, y3, degp, cstf):
    rpw = NP // 32
    return pl.kernel(
        _fin_body,
        out_type=[
            jax.ShapeDtypeStruct((NP, 8), jnp.float32),
            jax.ShapeDtypeStruct((NP, 8), jnp.float32),
        ],
        mesh=_mesh,
        compiler_params=_sc_params,
        scratch_types=[
            pltpu.VMEM((rpw,), jnp.float32),
            pltpu.VMEM((rpw,), jnp.float32),
            pltpu.VMEM((rpw,), jnp.float32),
            pltpu.VMEM((rpw, 8), jnp.float32),
            pltpu.VMEM((rpw, 8), jnp.float32),
            pltpu.VMEM((rpw, 8), jnp.float32),
            pltpu.VMEM((4, 16), jnp.float32),
            pltpu.VMEM((16,), jnp.float32),
            pltpu.VMEM((rpw, 8), jnp.float32),
            pltpu.VMEM((rpw, 8), jnp.float32),
        ],
    )(accp, y3, degp, cstf)


def _tca_body(x_ref, w1_ref, y_ref):
    y_ref[...] = jnp.dot(x_ref[...], w1_ref[...],
                         preferred_element_type=jnp.float32)


def _tca(x_pad, w1p):
    return pl.pallas_call(
        _tca_body,
        out_shape=jax.ShapeDtypeStruct((NP, 8), jnp.float32),
    )(x_pad, w1p)


def _const_block(b, w):
    bp = jnp.pad(b, (0, 8 - b.shape[0]))
    wp = jnp.pad(w, ((0, 4 - w.shape[0]), (0, 8 - w.shape[1])))
    rows = [jnp.tile(bp, 2)] + [jnp.tile(wp[k], 2) for k in range(4)]
    return jnp.stack(rows)


def kernel(x, edge_index, W1, b1, W2, b2, W3, b3, Wc, bc):
    ei3 = edge_index.astype(jnp.int32).reshape(2, NWIN, WIN)
    x_pad = jnp.pad(x, ((0, NP - N), (0, 0)))
    w1p = jnp.pad(W1, ((0, 0), (0, 4)))
    zeros1 = jnp.zeros((NP,), jnp.float32)
    zeros8 = jnp.zeros((NP, 8), jnp.float32)
    cst1 = _const_block(b1, W2)
    cst2 = _const_block(b2, W3)

    cstf = jnp.stack([
        jnp.tile(jnp.pad(b3, (0, 6)), 2),
        jnp.tile(jnp.pad(Wc[0], (0, 4)), 2),
        jnp.tile(jnp.pad(Wc[1], (0, 4)), 2),
        jnp.tile(jnp.pad(bc, (0, 4)), 2),
    ])

    xw1 = _tca(x_pad, w1p)
    degp = _deg_call(ei3, zeros1)

    acc1 = _mp1_call(ei3, xw1, degp, zeros8)
    acc2, y2 = _mp2_call(ei3, acc1, xw1, degp, cst1, zeros8)
    acc3, y3 = _mp2_call(ei3, acc2, y2, degp, cst2, zeros8, scale_prev=False)

    out8, h8 = _fin_call(acc3, y3, degp, cstf)
    return (out8[:N, :4], h8[:N, :2])


# speedup vs baseline: 71.1034x; 1.0920x over previous
import functools

import jax
import jax.numpy as jnp
from jax import lax
from jax.experimental import pallas as pl
from jax.experimental.pallas import tpu as pltpu
from jax.experimental.pallas import tpu_sc as plsc

N = 10000
E = 320000
NC = 2
NS = 16
NP = 10240
RPT = NP // NS
GPT = RPT // 2
WIN = 128
NWIN = E // WIN
WPW = 78
CHROWS = 13
NCHUNK = WPW // CHROWS
NEXTRA = NWIN - 32 * WPW

_mesh = plsc.VectorSubcoreMesh(
    core_axis_name="c", subcore_axis_name="s", num_cores=NC, num_subcores=NS
)
_sc_params = pltpu.CompilerParams(use_tc_tiling_on_sc=False,
                                  needs_layout_passes=False)


def _tanh16(v):
    av = jnp.abs(v)
    e = jnp.exp(av * -2.0)
    t = (1.0 - e) / (1.0 + e)
    return jnp.where(v < 0.0, -t, t)


def _fill_dis(g_d0, g_d1, g_dis):
    def body(j, carry):
        d = g_d0[pl.ds(16 * j, 16)] + g_d1[pl.ds(16 * j, 16)] + 1.0
        i = plsc.bitcast(d, jnp.int32)
        i = 0x5F3759DF - lax.shift_right_logical(i, 1)
        r = plsc.bitcast(i, jnp.float32)
        r = r * (1.5 - 0.5 * d * r * r)
        r = r * (1.5 - 0.5 * d * r * r)
        r = r * (1.5 - 0.5 * d * r * r)
        g_dis[pl.ds(16 * j, 16)] = r
        return carry

    lax.fori_loop(0, RPT // 16, body, 0, unroll=2)


def _edge_pass(w, ei_ref, y_sh, acc_sh, isrc_v, idst_v, rows_v,
               gsem_a, gsem_b, ssem):
    gsems = (gsem_a, gsem_b)

    pltpu.sync_copy(ei_ref.at[0, pl.ds(w * WPW, WPW)], isrc_v)
    pltpu.sync_copy(ei_ref.at[1, pl.ds(w * WPW, WPW)], idst_v)

    def chunk(i, carry):
        b = i * CHROWS
        gath = [None] * CHROWS
        gath[0] = pltpu.async_copy(y_sh.at[isrc_v.at[b]],
                                   rows_v.at[pl.ds(0, WIN)], gsems[0])
        scat = []
        for j in range(CHROWS):
            if j + 1 < CHROWS:
                gath[j + 1] = pltpu.async_copy(
                    y_sh.at[isrc_v.at[b + j + 1]],
                    rows_v.at[pl.ds((j + 1) * WIN, WIN)], gsems[(j + 1) % 2])
            gath[j].wait()
            scat.append(pltpu.async_copy(rows_v.at[pl.ds(j * WIN, WIN)],
                                         acc_sh.at[idst_v.at[b + j]], ssem,
                                         add=True))
        for d in scat:
            d.wait()
        return carry

    lax.fori_loop(0, NCHUNK, chunk, 0)

    @pl.when(w < NEXTRA)
    def _():
        r0 = 32 * WPW + w
        pltpu.sync_copy(ei_ref.at[0, pl.ds(r0, 1)], isrc_v.at[pl.ds(0, 1)])
        pltpu.sync_copy(ei_ref.at[1, pl.ds(r0, 1)], idst_v.at[pl.ds(0, 1)])
        pltpu.sync_copy(y_sh.at[isrc_v.at[0]], rows_v.at[pl.ds(0, WIN)])
        pltpu.sync_copy(rows_v.at[pl.ds(0, WIN)], acc_sh.at[idst_v.at[0]],
                        add=True)


def _deg_body(ei_ref, zeros_ref, out_ref, idx_v, ones_v, deg_sh, dsem):
    c = lax.axis_index("c")
    s = lax.axis_index("s")
    w = c * NS + s

    @pl.when(s == 0)
    def _():
        pltpu.sync_copy(zeros_ref, deg_sh)

    for k in range(WIN // 16):
        ones_v[pl.ds(k * 16, 16)] = jnp.ones((16,), jnp.float32)
    plsc.subcore_barrier()

    pltpu.sync_copy(ei_ref.at[1, pl.ds(w * WPW, WPW)], idx_v)

    def chunk(i, carry):
        b = i * CHROWS
        scat = [pltpu.async_copy(ones_v, deg_sh.at[idx_v.at[b + j]], dsem,
                                 add=True)
                for j in range(CHROWS)]
        for d in scat:
            d.wait()
        return carry

    lax.fori_loop(0, NCHUNK, chunk, 0)

    @pl.when(w < NEXTRA)
    def _():
        r0 = 32 * WPW + w
        pltpu.sync_copy(ei_ref.at[1, pl.ds(r0, 1)], idx_v.at[pl.ds(0, 1)])
        pltpu.sync_copy(ones_v, deg_sh.at[idx_v.at[0]], add=True)

    plsc.subcore_barrier()
    pltpu.sync_copy(deg_sh.at[pl.ds(s * RPT, RPT)],
                    out_ref.at[c, pl.ds(s * RPT, RPT)])


def _deg_call(ei3, zeros1):
    return pl.kernel(
        _deg_body,
        out_type=jax.ShapeDtypeStruct((NC, NP), jnp.float32),
        mesh=_mesh,
        compiler_params=_sc_params,
        scratch_types=[
            pltpu.VMEM((WPW, WIN), jnp.int32),
            pltpu.VMEM((WIN,), jnp.float32),
            pltpu.VMEM_SHARED((NP,), jnp.float32),
            pltpu.SemaphoreType.DMA,
        ],
    )(ei3, zeros1)


def _mp1_body(ei_ref, xw_ref, degp_ref, zeros_ref, out_ref,
              isrc_v, idst_v, rows_v, g_d0, g_d1, g_dis, g_in, g_y,
              y_sh, acc_sh, gsem_a, gsem_b, ssem):
    c = lax.axis_index("c")
    s = lax.axis_index("s")
    w = c * NS + s

    @pl.when(s == 0)
    def _():
        pltpu.sync_copy(zeros_ref, acc_sh)

    pltpu.sync_copy(degp_ref.at[0, pl.ds(s * RPT, RPT)], g_d0)
    pltpu.sync_copy(degp_ref.at[1, pl.ds(s * RPT, RPT)], g_d1)
    pltpu.sync_copy(xw_ref.at[pl.ds(s * RPT, RPT)], g_in)
    _fill_dis(g_d0, g_d1, g_dis)

    iota = lax.iota(jnp.int32, 16)
    rhalf = lax.shift_right_logical(iota, 3)
    col16 = jnp.bitwise_and(iota, 7)

    def grp(g, carry):
        rowi = rhalf + 2 * g
        db = plsc.load_gather(g_dis, (rowi,))
        xw = plsc.load_gather(g_in, (rowi, col16))
        plsc.store_scatter(g_y, (rowi, col16), db * xw)
        return carry

    lax.fori_loop(0, GPT, grp, 0, unroll=4)
    pltpu.sync_copy(g_y, y_sh.at[pl.ds(s * RPT, RPT)])
    plsc.subcore_barrier()
    _edge_pass(w, ei_ref, y_sh, acc_sh, isrc_v, idst_v, rows_v, gsem_a, gsem_b, ssem)
    plsc.subcore_barrier()
    pltpu.sync_copy(acc_sh.at[pl.ds(s * RPT, RPT)],
                    out_ref.at[c, pl.ds(s * RPT, RPT)])


def _mp1_call(ei3, xw1, degp, zeros8):
    return pl.kernel(
        _mp1_body,
        out_type=jax.ShapeDtypeStruct((NC, NP, 8), jnp.float32),
        mesh=_mesh,
        compiler_params=_sc_params,
        scratch_types=[
            pltpu.VMEM((WPW, WIN), jnp.int32),
            pltpu.VMEM((WPW, WIN), jnp.int32),
            pltpu.VMEM((CHROWS * WIN, 8), jnp.float32),
            pltpu.VMEM((RPT,), jnp.float32),
            pltpu.VMEM((RPT,), jnp.float32),
            pltpu.VMEM((RPT,), jnp.float32),
            pltpu.VMEM((RPT, 8), jnp.float32),
            pltpu.VMEM((RPT, 8), jnp.float32),
            pltpu.VMEM_SHARED((NP, 8), jnp.float32),
            pltpu.VMEM_SHARED((NP, 8), jnp.float32),
            pltpu.SemaphoreType.DMA,
            pltpu.SemaphoreType.DMA,
            pltpu.SemaphoreType.DMA,
        ],
    )(ei3, xw1, degp, zeros8)


def _glue_layer(g_a0, g_a1, g_prev, g_dis, g_c, g_y, g_t, g_dis8=None,
                scale_prev=False):
    iota = lax.iota(jnp.int32, 16)
    rhalf = lax.shift_right_logical(iota, 3)
    col16 = jnp.bitwise_and(iota, 7)
    rh8 = rhalf * 8
    bv = g_c[0]

    def grp(g, carry):
        rowi = rhalf + 2 * g
        db = plsc.load_gather(g_dis, (rowi,))
        prev = plsc.load_gather(g_prev, (rowi, col16))
        if scale_prev:
            prev = prev * db
        a = (plsc.load_gather(g_a0, (rowi, col16))
             + plsc.load_gather(g_a1, (rowi, col16))
             + prev)
        t = _tanh16(a * db + bv)
        acc = t.at[rh8].get(mode="promise_in_bounds") * g_c[1]
        acc = acc + t.at[rh8 + 1].get(mode="promise_in_bounds") * g_c[2]
        acc = acc + t.at[rh8 + 2].get(mode="promise_in_bounds") * g_c[3]
        acc = acc + t.at[rh8 + 3].get(mode="promise_in_bounds") * g_c[4]
        plsc.store_scatter(g_y, (rowi, col16), db * acc)
        if g_dis8 is not None:
            plsc.store_scatter(g_dis8, (rowi, col16), db)
        return carry

    lax.fori_loop(0, GPT, grp, 0, unroll=4)


def _mp2_body(scale_prev, ei_ref, accp_ref, prev_ref, degp_ref, cst_ref, zeros_ref,
              out_ref, ynext_ref,
              isrc_v, idst_v, rows_v, g_d0, g_d1, g_dis,
              g_a0, g_a1, g_prev, g_c, g_t, g_y, y_sh, acc_sh,
              gsem_a, gsem_b, ssem):
    c = lax.axis_index("c")
    s = lax.axis_index("s")
    w = c * NS + s

    @pl.when(s == 0)
    def _():
        pltpu.sync_copy(zeros_ref, acc_sh)

    pltpu.sync_copy(degp_ref.at[0, pl.ds(s * RPT, RPT)], g_d0)
    pltpu.sync_copy(degp_ref.at[1, pl.ds(s * RPT, RPT)], g_d1)
    pltpu.sync_copy(accp_ref.at[0, pl.ds(s * RPT, RPT)], g_a0)
    pltpu.sync_copy(accp_ref.at[1, pl.ds(s * RPT, RPT)], g_a1)
    pltpu.sync_copy(prev_ref.at[pl.ds(s * RPT, RPT)], g_prev)
    pltpu.sync_copy(cst_ref, g_c)
    _fill_dis(g_d0, g_d1, g_dis)
    _glue_layer(g_a0, g_a1, g_prev, g_dis, g_c, g_y, g_t, scale_prev=scale_prev)
    pltpu.sync_copy(g_y, y_sh.at[pl.ds(s * RPT, RPT)])

    @pl.when(c == 0)
    def _():
        pltpu.sync_copy(g_y, ynext_ref.at[pl.ds(s * RPT, RPT)])

    plsc.subcore_barrier()
    _edge_pass(w, ei_ref, y_sh, acc_sh, isrc_v, idst_v, rows_v, gsem_a, gsem_b, ssem)
    plsc.subcore_barrier()
    pltpu.sync_copy(acc_sh.at[pl.ds(s * RPT, RPT)],
                    out_ref.at[c, pl.ds(s * RPT, RPT)])


def _mp2_call(ei3, accp, prev_y, degp, cst, zeros8, scale_prev=True):
    return pl.kernel(
        functools.partial(_mp2_body, scale_prev),
        out_type=[
            jax.ShapeDtypeStruct((NC, NP, 8), jnp.float32),
            jax.ShapeDtypeStruct((NP, 8), jnp.float32),
        ],
        mesh=_mesh,
        compiler_params=_sc_params,
        scratch_types=[
            pltpu.VMEM((WPW, WIN), jnp.int32),
            pltpu.VMEM((WPW, WIN), jnp.int32),
            pltpu.VMEM((CHROWS * WIN, 8), jnp.float32),
            pltpu.VMEM((RPT,), jnp.float32),
            pltpu.VMEM((RPT,), jnp.float32),
            pltpu.VMEM((RPT,), jnp.float32),
            pltpu.VMEM((RPT, 8), jnp.float32),
            pltpu.VMEM((RPT, 8), jnp.float32),
            pltpu.VMEM((RPT, 8), jnp.float32),
            pltpu.VMEM((5, 16), jnp.float32),
            pltpu.VMEM((16,), jnp.float32),
            pltpu.VMEM((RPT, 8), jnp.float32),
            pltpu.VMEM_SHARED((NP, 8), jnp.float32),
            pltpu.VMEM_SHARED((NP, 8), jnp.float32),
            pltpu.SemaphoreType.DMA,
            pltpu.SemaphoreType.DMA,
            pltpu.SemaphoreType.DMA,
        ],
    )(ei3, accp, prev_y, degp, cst, zeros8)


def _fin_body(accp_ref, prev_ref, degp_ref, cst_ref, out8_ref, h8_ref,
              g_d0, g_d1, g_dis, g_a0, g_a1, g_prev, g_c, g_t, g_o, g_h):
    c = lax.axis_index("c")
    s = lax.axis_index("s")
    w = c * NS + s
    rpw = NP // 32
    r0 = w * rpw

    pltpu.sync_copy(degp_ref.at[0, pl.ds(r0, rpw)], g_d0)
    pltpu.sync_copy(degp_ref.at[1, pl.ds(r0, rpw)], g_d1)
    pltpu.sync_copy(accp_ref.at[0, pl.ds(r0, rpw)], g_a0)
    pltpu.sync_copy(accp_ref.at[1, pl.ds(r0, rpw)], g_a1)
    pltpu.sync_copy(prev_ref.at[pl.ds(r0, rpw)], g_prev)
    pltpu.sync_copy(cst_ref, g_c)

    def fill(j, carry):
        d = g_d0[pl.ds(16 * j, 16)] + g_d1[pl.ds(16 * j, 16)] + 1.0
        i = plsc.bitcast(d, jnp.int32)
        i = 0x5F3759DF - lax.shift_right_logical(i, 1)
        r = plsc.bitcast(i, jnp.float32)
        r = r * (1.5 - 0.5 * d * r * r)
        r = r * (1.5 - 0.5 * d * r * r)
        r = r * (1.5 - 0.5 * d * r * r)
        g_dis[pl.ds(16 * j, 16)] = r
        return carry

    lax.fori_loop(0, rpw // 16, fill, 0, unroll=2)

    iota = lax.iota(jnp.int32, 16)
    rhalf = lax.shift_right_logical(iota, 3)
    col16 = jnp.bitwise_and(iota, 7)
    rh8 = rhalf * 8
    bv = g_c[0]
    bcv = g_c[3]

    def grp(g, carry):
        rowi = rhalf + 2 * g
        db = plsc.load_gather(g_dis, (rowi,))
        a = (plsc.load_gather(g_a0, (rowi, col16))
             + plsc.load_gather(g_a1, (rowi, col16))
             + plsc.load_gather(g_prev, (rowi, col16)))
        t = _tanh16(a * db + bv)
        o = (t.at[rh8].get(mode="promise_in_bounds") * g_c[1]
             + t.at[rh8 + 1].get(mode="promise_in_bounds") * g_c[2] + bcv)
        plsc.store_scatter(g_o, (rowi, col16), o)
        plsc.store_scatter(g_h, (rowi, col16), t)
        return carry

    lax.fori_loop(0, rpw // 2, grp, 0, unroll=4)
    pltpu.sync_copy(g_o, out8_ref.at[pl.ds(r0, rpw)])
    pltpu.sync_copy(g_h, h8_ref.at[pl.ds(r0, rpw)])


def _fin_call(accp, y3, degp, cstf):
    rpw = NP // 32
    return pl.kernel(
        _fin_body,
        out_type=[
            jax.ShapeDtypeStruct((NP, 8), jnp.float32),
            jax.ShapeDtypeStruct((NP, 8), jnp.float32),
        ],
        mesh=_mesh,
        compiler_params=_sc_params,
        scratch_types=[
            pltpu.VMEM((rpw,), jnp.float32),
            pltpu.VMEM((rpw,), jnp.float32),
            pltpu.VMEM((rpw,), jnp.float32),
            pltpu.VMEM((rpw, 8), jnp.float32),
            pltpu.VMEM((rpw, 8), jnp.float32),
            pltpu.VMEM((rpw, 8), jnp.float32),
            pltpu.VMEM((4, 16), jnp.float32),
            pltpu.VMEM((16,), jnp.float32),
            pltpu.VMEM((rpw, 8), jnp.float32),
            pltpu.VMEM((rpw, 8), jnp.float32),
        ],
    )(accp, y3, degp, cstf)


def _tca_body(x_ref, w1_ref, y_ref):
    y_ref[...] = jnp.dot(x_ref[...], w1_ref[...],
                         preferred_element_type=jnp.float32)


def _tca(x_pad, w1p):
    return pl.pallas_call(
        _tca_body,
        out_shape=jax.ShapeDtypeStruct((NP, 8), jnp.float32),
    )(x_pad, w1p)


def _const_block(b, w):
    bp = jnp.pad(b, (0, 8 - b.shape[0]))
    wp = jnp.pad(w, ((0, 4 - w.shape[0]), (0, 8 - w.shape[1])))
    rows = [jnp.tile(bp, 2)] + [jnp.tile(wp[k], 2) for k in range(4)]
    return jnp.stack(rows)


def kernel(x, edge_index, W1, b1, W2, b2, W3, b3, Wc, bc):
    ei3 = edge_index.astype(jnp.int32).reshape(2, NWIN, WIN)
    x_pad = jnp.pad(x, ((0, NP - N), (0, 0)))
    w1p = jnp.pad(W1, ((0, 0), (0, 4)))
    zeros1 = jnp.zeros((NP,), jnp.float32)
    zeros8 = jnp.zeros((NP, 8), jnp.float32)
    cst1 = _const_block(b1, W2)
    cst2 = _const_block(b2, W3)

    cstf = jnp.stack([
        jnp.tile(jnp.pad(b3, (0, 6)), 2),
        jnp.tile(jnp.pad(Wc[0], (0, 4)), 2),
        jnp.tile(jnp.pad(Wc[1], (0, 4)), 2),
        jnp.tile(jnp.pad(bc, (0, 4)), 2),
    ])

    xw1 = _tca(x_pad, w1p)
    degp = _deg_call(ei3, zeros1)

    acc1 = _mp1_call(ei3, xw1, degp, zeros8)
    acc2, y2 = _mp2_call(ei3, acc1, xw1, degp, cst1, zeros8)
    acc3, y3 = _mp2_call(ei3, acc2, y2, degp, cst2, zeros8, scale_prev=False)

    out8, h8 = _fin_call(acc3, y3, degp, cstf)
    return (out8[:N, :4], h8[:N, :2])
